# Initial kernel scaffold; baseline (speedup 1.0000x reference)
import functools

import jax
import jax.numpy as jnp
from jax import lax
from jax.experimental import pallas as pl
from jax.experimental.pallas import tpu as pltpu
from jax.experimental.pallas import tpu_sc as plsc

N = 10000
E = 320000
D = 128
EPS = 1e-5

NC = 2
NS = 16
NW = NC * NS
EPW = E // NW
B = 125
NB = EPW // B
RPT = N // NS
RCH = 125
NRC = RPT // RCH


def _sc_segsum_body(x0_hbm, src_hbm, dst_hbm, out_hbm,
                    src_v, dst_v, rows0, rows1, acc_sh, sem0, sem1):
    cid = lax.axis_index("c")
    sid = lax.axis_index("s")
    w = cid * NS + sid

    for t in range(3):
        for r in range(NRC):
            rs = sid * RPT + r * RCH
            pltpu.sync_copy(x0_hbm.at[pl.ds(rs, RCH)], rows0)
            pltpu.sync_copy(rows0, acc_sh.at[pl.ds(rs, RCH)])
        plsc.subcore_barrier()

        pltpu.sync_copy(src_hbm.at[t, w], src_v)
        pltpu.sync_copy(dst_hbm.at[t, w], dst_v)

        def body(i, carry):
            j0 = 2 * i
            j1 = 2 * i + 1
            cp0 = pltpu.async_copy(x0_hbm.at[src_v.at[j0]], rows0, sem0)
            cp1 = pltpu.async_copy(x0_hbm.at[src_v.at[j1]], rows1, sem1)
            cp0.wait()
            pltpu.sync_copy(rows0, acc_sh.at[dst_v.at[j0]], add=True)
            cp1.wait()
            pltpu.sync_copy(rows1, acc_sh.at[dst_v.at[j1]], add=True)
            return carry

        lax.fori_loop(0, NB // 2, body, 0)
        plsc.subcore_barrier()

        for r in range(NRC):
            rs = sid * RPT + r * RCH
            pltpu.sync_copy(acc_sh.at[pl.ds(rs, RCH)], rows0)
            pltpu.sync_copy(rows0, out_hbm.at[t, cid, pl.ds(rs, RCH)])
        plsc.subcore_barrier()


_sc_segsum = functools.partial(
    pl.kernel,
    out_type=jax.ShapeDtypeStruct((3, NC, N, D), jnp.float32),
    mesh=plsc.VectorSubcoreMesh(core_axis_name="c", subcore_axis_name="s",
                                num_cores=NC, num_subcores=NS),
    scratch_types=[
        pltpu.VMEM((NB, B), jnp.int32),
        pltpu.VMEM((NB, B), jnp.int32),
        pltpu.VMEM((B, D), jnp.float32),
        pltpu.VMEM((B, D), jnp.float32),
        pltpu.VMEM_SHARED((N, D), jnp.float32),
        pltpu.SemaphoreType.DMA,
        pltpu.SemaphoreType.DMA,
    ],
)(_sc_segsum_body)


def _bn_in_body(x_ref, g_ref, b_ref, o_ref):
    xv = x_ref[...]
    m = jnp.mean(xv, axis=0, keepdims=True)
    v = jnp.mean(jnp.square(xv - m), axis=0, keepdims=True)
    o_ref[...] = (xv - m) * lax.rsqrt(v + EPS) * g_ref[...] + b_ref[...]


def _bn_in(x, g, b):
    return pl.pallas_call(
        _bn_in_body,
        out_shape=jax.ShapeDtypeStruct((N, D), jnp.float32),
    )(x, g.reshape(1, D), b.reshape(1, D))


RB = 2000
G = N // RB


def _c1_body(x0_ref, p_ref, w1_ref, b1_ref, u_ref, s_ref, acc):
    i = pl.program_id(0)

    @pl.when(i == 0)
    def _():
        acc[...] = jnp.zeros_like(acc)

    for t in range(3):
        h = p_ref[t, 0] + p_ref[t, 1] - x0_ref[...]
        u = jnp.dot(h, w1_ref[t], preferred_element_type=jnp.float32) + b1_ref[t]
        u_ref[t] = u
        acc[2 * t:2 * t + 1, :] += jnp.sum(u, axis=0, keepdims=True)
        acc[2 * t + 1:2 * t + 2, :] += jnp.sum(u * u, axis=0, keepdims=True)

    @pl.when(i == G - 1)
    def _():
        s_ref[...] = acc[...]


def _c1(x0, p, W1, b1):
    return pl.pallas_call(
        _c1_body,
        grid=(G,),
        in_specs=[
            pl.BlockSpec((RB, D), lambda i: (i, 0)),
            pl.BlockSpec((3, NC, RB, D), lambda i: (0, 0, i, 0)),
            pl.BlockSpec((3, D, D), lambda i: (0, 0, 0)),
            pl.BlockSpec((3, 1, D), lambda i: (0, 0, 0)),
        ],
        out_specs=[
            pl.BlockSpec((3, RB, D), lambda i: (0, i, 0)),
            pl.BlockSpec((6, D), lambda i: (0, 0)),
        ],
        out_shape=[
            jax.ShapeDtypeStruct((3, N, D), jnp.float32),
            jax.ShapeDtypeStruct((6, D), jnp.float32),
        ],
        scratch_shapes=[pltpu.VMEM((6, D), jnp.float32)],
    )(x0, p, W1, b1.reshape(3, 1, D))


def _c2_body(u_ref, s1_ref, g1_ref, b1_ref, w2_ref, b2_ref, v_ref, s_ref, acc):
    i = pl.program_id(0)

    @pl.when(i == 0)
    def _():
        acc[...] = jnp.zeros_like(acc)

    for t in range(3):
        m = s1_ref[2 * t:2 * t + 1, :] * (1.0 / N)
        var = s1_ref[2 * t + 1:2 * t + 2, :] * (1.0 / N) - m * m
        a = (u_ref[t] - m) * lax.rsqrt(var + EPS) * g1_ref[t] + b1_ref[t]
        a = jnp.maximum(a, 0.0)
        v = jnp.dot(a, w2_ref[t], preferred_element_type=jnp.float32) + b2_ref[t]
        v_ref[t] = v
        acc[2 * t:2 * t + 1, :] += jnp.sum(v, axis=0, keepdims=True)
        acc[2 * t + 1:2 * t + 2, :] += jnp.sum(v * v, axis=0, keepdims=True)

    @pl.when(i == G - 1)
    def _():
        s_ref[...] = acc[...]


def _c2(u, s1, bn1_g, bn1_b, W2, b2):
    return pl.pallas_call(
        _c2_body,
        grid=(G,),
        in_specs=[
            pl.BlockSpec((3, RB, D), lambda i: (0, i, 0)),
            pl.BlockSpec((6, D), lambda i: (0, 0)),
            pl.BlockSpec((3, 1, D), lambda i: (0, 0, 0)),
            pl.BlockSpec((3, 1, D), lambda i: (0, 0, 0)),
            pl.BlockSpec((3, D, D), lambda i: (0, 0, 0)),
            pl.BlockSpec((3, 1, D), lambda i: (0, 0, 0)),
        ],
        out_specs=[
            pl.BlockSpec((3, RB, D), lambda i: (0, i, 0)),
            pl.BlockSpec((6, D), lambda i: (0, 0)),
        ],
        out_shape=[
            jax.ShapeDtypeStruct((3, N, D), jnp.float32),
            jax.ShapeDtypeStruct((6, D), jnp.float32),
        ],
        scratch_shapes=[pltpu.VMEM((6, D), jnp.float32)],
    )(u, s1, bn1_g.reshape(3, 1, D), bn1_b.reshape(3, 1, D), W2,
      b2.reshape(3, 1, D))


def _c3_body(v_ref, s2_ref, go_ref, bo_ref, x0_ref,
             wq_ref, bq_ref, wk_ref, bk_ref, wv_ref, bv_ref, o_ref):
    q = jnp.tanh(jnp.dot(x0_ref[...], wq_ref[...],
                         preferred_element_type=jnp.float32) + bq_ref[...])
    scores = []
    vals = []
    for t in range(3):
        m = s2_ref[2 * t:2 * t + 1, :] * (1.0 / N)
        var = s2_ref[2 * t + 1:2 * t + 2, :] * (1.0 / N) - m * m
        e = jnp.tanh((v_ref[t] - m) * lax.rsqrt(var + EPS) * go_ref[t]
                     + bo_ref[t])
        k = jnp.tanh(jnp.dot(e, wk_ref[...],
                             preferred_element_type=jnp.float32) + bk_ref[...])
        vv = jnp.tanh(jnp.dot(e, wv_ref[...],
                              preferred_element_type=jnp.float32) + bv_ref[...])
        scores.append(jnp.sum(k * q, axis=1, keepdims=True))
        vals.append(vv)
    smax = jnp.maximum(jnp.maximum(scores[0], scores[1]), scores[2])
    ew = [jnp.exp(s - smax) for s in scores]
    z = ew[0] + ew[1] + ew[2]
    o_ref[...] = (ew[0] * vals[0] + ew[1] * vals[1] + ew[2] * vals[2]) / z


def _c3(v, s2, bno_g, bno_b, x0, Wq, bq, Wk, bk, Wv, bv):
    return pl.pallas_call(
        _c3_body,
        grid=(G,),
        in_specs=[
            pl.BlockSpec((3, RB, D), lambda i: (0, i, 0)),
            pl.BlockSpec((6, D), lambda i: (0, 0)),
            pl.BlockSpec((3, 1, D), lambda i: (0, 0, 0)),
            pl.BlockSpec((3, 1, D), lambda i: (0, 0, 0)),
            pl.BlockSpec((RB, D), lambda i: (i, 0)),
            pl.BlockSpec((D, D), lambda i: (0, 0)),
            pl.BlockSpec((1, D), lambda i: (0, 0)),
            pl.BlockSpec((D, D), lambda i: (0, 0)),
            pl.BlockSpec((1, D), lambda i: (0, 0)),
            pl.BlockSpec((D, D), lambda i: (0, 0)),
            pl.BlockSpec((1, D), lambda i: (0, 0)),
        ],
        out_specs=pl.BlockSpec((RB, D), lambda i: (i, 0)),
        out_shape=jax.ShapeDtypeStruct((N, D), jnp.float32),
    )(v, s2, bno_g.reshape(3, 1, D), bno_b.reshape(3, 1, D), x0,
      Wq, bq.reshape(1, D), Wk, bk.reshape(1, D), Wv, bv.reshape(1, D))


def kernel(x, edge_index_p, edge_index_s, edge_index_v, g_in, b_in,
           W1, b1, bn1_g, bn1_b, W2, b2, bno_g, bno_b,
           Wq, bq, Wk, bk, Wv, bv):
    src = jnp.stack([edge_index_p[0], edge_index_s[0], edge_index_v[0]])
    dst = jnp.stack([edge_index_p[1], edge_index_s[1], edge_index_v[1]])
    src = src.reshape(3, NW, NB, B)
    dst = dst.reshape(3, NW, NB, B)

    x0 = _bn_in(x, g_in, b_in)
    p = _sc_segsum(x0, src, dst)
    u, s1 = _c1(x0, p, W1, b1)
    v, s2 = _c2(u, s1, bn1_g, bn1_b, W2, b2)
    return _c3(v, s2, bno_g, bno_b, x0, Wq, bq, Wk, bk, Wv, bv)


# speedup vs baseline: 5.9609x; 5.9609x over previous
import functools

import jax
import jax.numpy as jnp
from jax import lax
from jax.experimental import pallas as pl
from jax.experimental.pallas import tpu as pltpu
from jax.experimental.pallas import tpu_sc as plsc

N = 10000
E = 320000
D = 128
EPS = 1e-5

NC = 2
NS = 16
NW = NC * NS
EPW = E // NW
B = 125
NB = EPW // B
NP = 10240
RPT = NP // NS
RCH = 32
NRC = RPT // RCH
NBC = 8
NCH = NB // NBC


def _sc_segsum_body(x0_hbm, src_hbm, dst_hbm, out_hbm,
                    src_v, dst_v, rows0, rows1, stage, acc_sh, sem0, sem1):
    cid = lax.axis_index("c")
    sid = lax.axis_index("s")
    w = cid * NS + sid

    for t in range(3):
        for r in range(NRC):
            rs = sid * RPT + r * RCH
            pltpu.sync_copy(x0_hbm.at[pl.ds(rs, RCH)], stage)
            pltpu.sync_copy(stage, acc_sh.at[pl.ds(rs, RCH)])
        plsc.subcore_barrier()

        def chunk_body(c, carry):
            cb = pl.multiple_of(c * NBC, 8)
            pltpu.sync_copy(src_hbm.at[t, w, pl.ds(cb, NBC)], src_v)
            pltpu.sync_copy(dst_hbm.at[t, w, pl.ds(cb, NBC)], dst_v)

            def body(i, carry2):
                j0 = 2 * i
                j1 = 2 * i + 1
                cp0 = pltpu.async_copy(x0_hbm.at[src_v.at[j0]], rows0, sem0)
                cp1 = pltpu.async_copy(x0_hbm.at[src_v.at[j1]], rows1, sem1)
                cp0.wait()
                pltpu.sync_copy(rows0, acc_sh.at[dst_v.at[j0]], add=True)
                cp1.wait()
                pltpu.sync_copy(rows1, acc_sh.at[dst_v.at[j1]], add=True)
                return carry2

            lax.fori_loop(0, NBC // 2, body, 0)
            return carry

        lax.fori_loop(0, NCH, chunk_body, 0)
        plsc.subcore_barrier()

        for r in range(NRC):
            rs = sid * RPT + r * RCH
            pltpu.sync_copy(acc_sh.at[pl.ds(rs, RCH)], stage)
            pltpu.sync_copy(stage, out_hbm.at[t, cid, pl.ds(rs, RCH)])
        plsc.subcore_barrier()


_sc_segsum = functools.partial(
    pl.kernel,
    out_type=jax.ShapeDtypeStruct((3, NC, NP, D), jnp.float32),
    mesh=plsc.VectorSubcoreMesh(core_axis_name="c", subcore_axis_name="s",
                                num_cores=NC, num_subcores=NS),
    scratch_types=[
        pltpu.VMEM((NBC, B), jnp.int32),
        pltpu.VMEM((NBC, B), jnp.int32),
        pltpu.VMEM((B, D), jnp.float32),
        pltpu.VMEM((B, D), jnp.float32),
        pltpu.VMEM((RCH, D), jnp.float32),
        pltpu.VMEM_SHARED((NP, D), jnp.float32),
        pltpu.SemaphoreType.DMA,
        pltpu.SemaphoreType.DMA,
    ],
)(_sc_segsum_body)


def _bn_in_body(x_ref, g_ref, b_ref, o_ref):
    xv = x_ref[...]
    m = jnp.mean(xv, axis=0, keepdims=True)
    v = jnp.mean(jnp.square(xv - m), axis=0, keepdims=True)
    x0 = (xv - m) * lax.rsqrt(v + EPS) * g_ref[...] + b_ref[...]
    o_ref[...] = jnp.concatenate(
        [x0, jnp.zeros((NP - N, D), jnp.float32)], axis=0)


def _bn_in(x, g, b):
    return pl.pallas_call(
        _bn_in_body,
        out_shape=jax.ShapeDtypeStruct((NP, D), jnp.float32),
    )(x, g.reshape(1, D), b.reshape(1, D))


RB = 2000
G = N // RB


def _c1_body(x0_ref, p_ref, w1_ref, b1_ref, u_ref, s_ref, acc):
    i = pl.program_id(0)

    @pl.when(i == 0)
    def _():
        acc[...] = jnp.zeros_like(acc)

    for t in range(3):
        h = p_ref[t, 0] + p_ref[t, 1] - x0_ref[...]
        u = jnp.dot(h, w1_ref[t], preferred_element_type=jnp.float32) + b1_ref[t]
        u_ref[t] = u
        acc[2 * t:2 * t + 1, :] += jnp.sum(u, axis=0, keepdims=True)
        acc[2 * t + 1:2 * t + 2, :] += jnp.sum(u * u, axis=0, keepdims=True)

    @pl.when(i == G - 1)
    def _():
        s_ref[...] = acc[...]


def _c1(x0, p, W1, b1):
    return pl.pallas_call(
        _c1_body,
        grid=(G,),
        in_specs=[
            pl.BlockSpec((RB, D), lambda i: (i, 0)),
            pl.BlockSpec((3, NC, RB, D), lambda i: (0, 0, i, 0)),
            pl.BlockSpec((3, D, D), lambda i: (0, 0, 0)),
            pl.BlockSpec((3, 1, D), lambda i: (0, 0, 0)),
        ],
        out_specs=[
            pl.BlockSpec((3, RB, D), lambda i: (0, i, 0)),
            pl.BlockSpec((6, D), lambda i: (0, 0)),
        ],
        out_shape=[
            jax.ShapeDtypeStruct((3, N, D), jnp.float32),
            jax.ShapeDtypeStruct((6, D), jnp.float32),
        ],
        scratch_shapes=[pltpu.VMEM((6, D), jnp.float32)],
    )(x0, p, W1, b1.reshape(3, 1, D))


def _c2_body(u_ref, s1_ref, g1_ref, b1_ref, w2_ref, b2_ref, v_ref, s_ref, acc):
    i = pl.program_id(0)

    @pl.when(i == 0)
    def _():
        acc[...] = jnp.zeros_like(acc)

    for t in range(3):
        m = s1_ref[2 * t:2 * t + 1, :] * (1.0 / N)
        var = s1_ref[2 * t + 1:2 * t + 2, :] * (1.0 / N) - m * m
        a = (u_ref[t] - m) * lax.rsqrt(var + EPS) * g1_ref[t] + b1_ref[t]
        a = jnp.maximum(a, 0.0)
        v = jnp.dot(a, w2_ref[t], preferred_element_type=jnp.float32) + b2_ref[t]
        v_ref[t] = v
        acc[2 * t:2 * t + 1, :] += jnp.sum(v, axis=0, keepdims=True)
        acc[2 * t + 1:2 * t + 2, :] += jnp.sum(v * v, axis=0, keepdims=True)

    @pl.when(i == G - 1)
    def _():
        s_ref[...] = acc[...]


def _c2(u, s1, bn1_g, bn1_b, W2, b2):
    return pl.pallas_call(
        _c2_body,
        grid=(G,),
        in_specs=[
            pl.BlockSpec((3, RB, D), lambda i: (0, i, 0)),
            pl.BlockSpec((6, D), lambda i: (0, 0)),
            pl.BlockSpec((3, 1, D), lambda i: (0, 0, 0)),
            pl.BlockSpec((3, 1, D), lambda i: (0, 0, 0)),
            pl.BlockSpec((3, D, D), lambda i: (0, 0, 0)),
            pl.BlockSpec((3, 1, D), lambda i: (0, 0, 0)),
        ],
        out_specs=[
            pl.BlockSpec((3, RB, D), lambda i: (0, i, 0)),
            pl.BlockSpec((6, D), lambda i: (0, 0)),
        ],
        out_shape=[
            jax.ShapeDtypeStruct((3, N, D), jnp.float32),
            jax.ShapeDtypeStruct((6, D), jnp.float32),
        ],
        scratch_shapes=[pltpu.VMEM((6, D), jnp.float32)],
    )(u, s1, bn1_g.reshape(3, 1, D), bn1_b.reshape(3, 1, D), W2,
      b2.reshape(3, 1, D))


def _c3_body(v_ref, s2_ref, go_ref, bo_ref, x0_ref,
             wq_ref, bq_ref, wk_ref, bk_ref, wv_ref, bv_ref, o_ref):
    q = jnp.tanh(jnp.dot(x0_ref[...], wq_ref[...],
                         preferred_element_type=jnp.float32) + bq_ref[...])
    scores = []
    vals = []
    for t in range(3):
        m = s2_ref[2 * t:2 * t + 1, :] * (1.0 / N)
        var = s2_ref[2 * t + 1:2 * t + 2, :] * (1.0 / N) - m * m
        e = jnp.tanh((v_ref[t] - m) * lax.rsqrt(var + EPS) * go_ref[t]
                     + bo_ref[t])
        k = jnp.tanh(jnp.dot(e, wk_ref[...],
                             preferred_element_type=jnp.float32) + bk_ref[...])
        vv = jnp.tanh(jnp.dot(e, wv_ref[...],
                              preferred_element_type=jnp.float32) + bv_ref[...])
        scores.append(jnp.sum(k * q, axis=1, keepdims=True))
        vals.append(vv)
    smax = jnp.maximum(jnp.maximum(scores[0], scores[1]), scores[2])
    ew = [jnp.exp(s - smax) for s in scores]
    z = ew[0] + ew[1] + ew[2]
    o_ref[...] = (ew[0] * vals[0] + ew[1] * vals[1] + ew[2] * vals[2]) / z


def _c3(v, s2, bno_g, bno_b, x0, Wq, bq, Wk, bk, Wv, bv):
    return pl.pallas_call(
        _c3_body,
        grid=(G,),
        in_specs=[
            pl.BlockSpec((3, RB, D), lambda i: (0, i, 0)),
            pl.BlockSpec((6, D), lambda i: (0, 0)),
            pl.BlockSpec((3, 1, D), lambda i: (0, 0, 0)),
            pl.BlockSpec((3, 1, D), lambda i: (0, 0, 0)),
            pl.BlockSpec((RB, D), lambda i: (i, 0)),
            pl.BlockSpec((D, D), lambda i: (0, 0)),
            pl.BlockSpec((1, D), lambda i: (0, 0)),
            pl.BlockSpec((D, D), lambda i: (0, 0)),
            pl.BlockSpec((1, D), lambda i: (0, 0)),
            pl.BlockSpec((D, D), lambda i: (0, 0)),
            pl.BlockSpec((1, D), lambda i: (0, 0)),
        ],
        out_specs=pl.BlockSpec((RB, D), lambda i: (i, 0)),
        out_shape=jax.ShapeDtypeStruct((N, D), jnp.float32),
    )(v, s2, bno_g.reshape(3, 1, D), bno_b.reshape(3, 1, D), x0,
      Wq, bq.reshape(1, D), Wk, bk.reshape(1, D), Wv, bv.reshape(1, D))


def kernel(x, edge_index_p, edge_index_s, edge_index_v, g_in, b_in,
           W1, b1, bn1_g, bn1_b, W2, b2, bno_g, bno_b,
           Wq, bq, Wk, bk, Wv, bv):
    src = jnp.stack([edge_index_p[0], edge_index_s[0], edge_index_v[0]])
    dst = jnp.stack([edge_index_p[1], edge_index_s[1], edge_index_v[1]])
    src = src.reshape(3, NW, NB, B)
    dst = dst.reshape(3, NW, NB, B)

    x0 = _bn_in(x, g_in, b_in)
    p = _sc_segsum(x0, src, dst)
    u, s1 = _c1(x0, p, W1, b1)
    v, s2 = _c2(u, s1, bn1_g, bn1_b, W2, b2)
    return _c3(v, s2, bno_g, bno_b, x0, Wq, bq, Wk, bk, Wv, bv)


# speedup vs baseline: 7.2591x; 1.2178x over previous
import functools

import jax
import jax.numpy as jnp
from jax import lax
from jax.experimental import pallas as pl
from jax.experimental.pallas import tpu as pltpu
from jax.experimental.pallas import tpu_sc as plsc

N = 10000
E = 320000
D = 128
EPS = 1e-5

NC = 2
NS = 16
NW = NC * NS
EPW = E // NW
B = 100
NB = EPW // B
NBC = 10
NCH = NB // NBC
NP = 10240
RPT = NP // NS
RCH = 32
NRC = RPT // RCH


def _sc_segsum_body(x0_hbm, src_hbm, dst_hbm, out_hbm,
                    src_v, dst_v, rows, stage, acc_sh, gsem, ssem):
    cid = lax.axis_index("c")
    sid = lax.axis_index("s")
    w = cid * NS + sid

    for t in range(3):
        for r in range(NRC):
            rs = sid * RPT + r * RCH
            pltpu.sync_copy(x0_hbm.at[pl.ds(rs, RCH)], stage)
            pltpu.sync_copy(stage, acc_sh.at[pl.ds(rs, RCH)])
        plsc.subcore_barrier()

        def chunk_body(c, carry):
            pltpu.sync_copy(src_hbm.at[t, w, c], src_v)
            pltpu.sync_copy(dst_hbm.at[t, w, c], dst_v)

            cp = {}
            sc = {}
            cp[0] = pltpu.async_copy(x0_hbm.at[src_v.at[0]], rows.at[0],
                                     gsem.at[0])
            cp[1] = pltpu.async_copy(x0_hbm.at[src_v.at[1]], rows.at[1],
                                     gsem.at[1])
            for j in range(NBC):
                cp[j].wait()
                sc[j] = pltpu.async_copy(rows.at[j % 3],
                                         acc_sh.at[dst_v.at[j]],
                                         ssem.at[j % 3], add=True)
                jn = j + 2
                if jn < NBC:
                    if jn >= 3:
                        sc[jn - 3].wait()
                    cp[jn] = pltpu.async_copy(x0_hbm.at[src_v.at[jn]],
                                              rows.at[jn % 3],
                                              gsem.at[jn % 3])
            for j in range(NBC - 3, NBC):
                sc[j].wait()
            return carry

        lax.fori_loop(0, NCH, chunk_body, 0)
        plsc.subcore_barrier()

        for r in range(NRC):
            rs = sid * RPT + r * RCH
            pltpu.sync_copy(acc_sh.at[pl.ds(rs, RCH)], stage)
            pltpu.sync_copy(stage, out_hbm.at[t, cid, pl.ds(rs, RCH)])
        plsc.subcore_barrier()


_sc_segsum = functools.partial(
    pl.kernel,
    out_type=jax.ShapeDtypeStruct((3, NC, NP, D), jnp.float32),
    mesh=plsc.VectorSubcoreMesh(core_axis_name="c", subcore_axis_name="s",
                                num_cores=NC, num_subcores=NS),
    scratch_types=[
        pltpu.VMEM((NBC, B), jnp.int32),
        pltpu.VMEM((NBC, B), jnp.int32),
        pltpu.VMEM((3, B, D), jnp.float32),
        pltpu.VMEM((RCH, D), jnp.float32),
        pltpu.VMEM_SHARED((NP, D), jnp.float32),
        pltpu.SemaphoreType.DMA((3,)),
        pltpu.SemaphoreType.DMA((3,)),
    ],
)(_sc_segsum_body)


def _bn_in_body(x_ref, g_ref, b_ref, o_ref):
    xv = x_ref[...]
    m = jnp.mean(xv, axis=0, keepdims=True)
    v = jnp.mean(jnp.square(xv - m), axis=0, keepdims=True)
    x0 = (xv - m) * lax.rsqrt(v + EPS) * g_ref[...] + b_ref[...]
    o_ref[...] = jnp.concatenate(
        [x0, jnp.zeros((NP - N, D), jnp.float32)], axis=0)


def _bn_in(x, g, b):
    return pl.pallas_call(
        _bn_in_body,
        out_shape=jax.ShapeDtypeStruct((NP, D), jnp.float32),
    )(x, g.reshape(1, D), b.reshape(1, D))


RB = 2000
G = N // RB


def _c1_body(x0_ref, p_ref, w1_ref, b1_ref, u_ref, s_ref, acc):
    i = pl.program_id(0)

    @pl.when(i == 0)
    def _():
        acc[...] = jnp.zeros_like(acc)

    for t in range(3):
        h = p_ref[t, 0] + p_ref[t, 1] - x0_ref[...]
        u = jnp.dot(h, w1_ref[t], preferred_element_type=jnp.float32) + b1_ref[t]
        u_ref[t] = u
        acc[2 * t:2 * t + 1, :] += jnp.sum(u, axis=0, keepdims=True)
        acc[2 * t + 1:2 * t + 2, :] += jnp.sum(u * u, axis=0, keepdims=True)

    @pl.when(i == G - 1)
    def _():
        s_ref[...] = acc[...]


def _c1(x0, p, W1, b1):
    return pl.pallas_call(
        _c1_body,
        grid=(G,),
        in_specs=[
            pl.BlockSpec((RB, D), lambda i: (i, 0)),
            pl.BlockSpec((3, NC, RB, D), lambda i: (0, 0, i, 0)),
            pl.BlockSpec((3, D, D), lambda i: (0, 0, 0)),
            pl.BlockSpec((3, 1, D), lambda i: (0, 0, 0)),
        ],
        out_specs=[
            pl.BlockSpec((3, RB, D), lambda i: (0, i, 0)),
            pl.BlockSpec((6, D), lambda i: (0, 0)),
        ],
        out_shape=[
            jax.ShapeDtypeStruct((3, N, D), jnp.float32),
            jax.ShapeDtypeStruct((6, D), jnp.float32),
        ],
        scratch_shapes=[pltpu.VMEM((6, D), jnp.float32)],
    )(x0, p, W1, b1.reshape(3, 1, D))


def _c2_body(u_ref, s1_ref, g1_ref, b1_ref, w2_ref, b2_ref, v_ref, s_ref, acc):
    i = pl.program_id(0)

    @pl.when(i == 0)
    def _():
        acc[...] = jnp.zeros_like(acc)

    for t in range(3):
        m = s1_ref[2 * t:2 * t + 1, :] * (1.0 / N)
        var = s1_ref[2 * t + 1:2 * t + 2, :] * (1.0 / N) - m * m
        a = (u_ref[t] - m) * lax.rsqrt(var + EPS) * g1_ref[t] + b1_ref[t]
        a = jnp.maximum(a, 0.0)
        v = jnp.dot(a, w2_ref[t], preferred_element_type=jnp.float32) + b2_ref[t]
        v_ref[t] = v
        acc[2 * t:2 * t + 1, :] += jnp.sum(v, axis=0, keepdims=True)
        acc[2 * t + 1:2 * t + 2, :] += jnp.sum(v * v, axis=0, keepdims=True)

    @pl.when(i == G - 1)
    def _():
        s_ref[...] = acc[...]


def _c2(u, s1, bn1_g, bn1_b, W2, b2):
    return pl.pallas_call(
        _c2_body,
        grid=(G,),
        in_specs=[
            pl.BlockSpec((3, RB, D), lambda i: (0, i, 0)),
            pl.BlockSpec((6, D), lambda i: (0, 0)),
            pl.BlockSpec((3, 1, D), lambda i: (0, 0, 0)),
            pl.BlockSpec((3, 1, D), lambda i: (0, 0, 0)),
            pl.BlockSpec((3, D, D), lambda i: (0, 0, 0)),
            pl.BlockSpec((3, 1, D), lambda i: (0, 0, 0)),
        ],
        out_specs=[
            pl.BlockSpec((3, RB, D), lambda i: (0, i, 0)),
            pl.BlockSpec((6, D), lambda i: (0, 0)),
        ],
        out_shape=[
            jax.ShapeDtypeStruct((3, N, D), jnp.float32),
            jax.ShapeDtypeStruct((6, D), jnp.float32),
        ],
        scratch_shapes=[pltpu.VMEM((6, D), jnp.float32)],
    )(u, s1, bn1_g.reshape(3, 1, D), bn1_b.reshape(3, 1, D), W2,
      b2.reshape(3, 1, D))


def _c3_body(v_ref, s2_ref, go_ref, bo_ref, x0_ref,
             wq_ref, bq_ref, wk_ref, bk_ref, wv_ref, bv_ref, o_ref):
    q = jnp.tanh(jnp.dot(x0_ref[...], wq_ref[...],
                         preferred_element_type=jnp.float32) + bq_ref[...])
    scores = []
    vals = []
    for t in range(3):
        m = s2_ref[2 * t:2 * t + 1, :] * (1.0 / N)
        var = s2_ref[2 * t + 1:2 * t + 2, :] * (1.0 / N) - m * m
        e = jnp.tanh((v_ref[t] - m) * lax.rsqrt(var + EPS) * go_ref[t]
                     + bo_ref[t])
        k = jnp.tanh(jnp.dot(e, wk_ref[...],
                             preferred_element_type=jnp.float32) + bk_ref[...])
        vv = jnp.tanh(jnp.dot(e, wv_ref[...],
                              preferred_element_type=jnp.float32) + bv_ref[...])
        scores.append(jnp.sum(k * q, axis=1, keepdims=True))
        vals.append(vv)
    smax = jnp.maximum(jnp.maximum(scores[0], scores[1]), scores[2])
    ew = [jnp.exp(s - smax) for s in scores]
    z = ew[0] + ew[1] + ew[2]
    o_ref[...] = (ew[0] * vals[0] + ew[1] * vals[1] + ew[2] * vals[2]) / z


def _c3(v, s2, bno_g, bno_b, x0, Wq, bq, Wk, bk, Wv, bv):
    return pl.pallas_call(
        _c3_body,
        grid=(G,),
        in_specs=[
            pl.BlockSpec((3, RB, D), lambda i: (0, i, 0)),
            pl.BlockSpec((6, D), lambda i: (0, 0)),
            pl.BlockSpec((3, 1, D), lambda i: (0, 0, 0)),
            pl.BlockSpec((3, 1, D), lambda i: (0, 0, 0)),
            pl.BlockSpec((RB, D), lambda i: (i, 0)),
            pl.BlockSpec((D, D), lambda i: (0, 0)),
            pl.BlockSpec((1, D), lambda i: (0, 0)),
            pl.BlockSpec((D, D), lambda i: (0, 0)),
            pl.BlockSpec((1, D), lambda i: (0, 0)),
            pl.BlockSpec((D, D), lambda i: (0, 0)),
            pl.BlockSpec((1, D), lambda i: (0, 0)),
        ],
        out_specs=pl.BlockSpec((RB, D), lambda i: (i, 0)),
        out_shape=jax.ShapeDtypeStruct((N, D), jnp.float32),
    )(v, s2, bno_g.reshape(3, 1, D), bno_b.reshape(3, 1, D), x0,
      Wq, bq.reshape(1, D), Wk, bk.reshape(1, D), Wv, bv.reshape(1, D))


def kernel(x, edge_index_p, edge_index_s, edge_index_v, g_in, b_in,
           W1, b1, bn1_g, bn1_b, W2, b2, bno_g, bno_b,
           Wq, bq, Wk, bk, Wv, bv):
    src = jnp.stack([edge_index_p[0], edge_index_s[0], edge_index_v[0]])
    dst = jnp.stack([edge_index_p[1], edge_index_s[1], edge_index_v[1]])
    src = src.reshape(3, NW, NCH, NBC, B)
    dst = dst.reshape(3, NW, NCH, NBC, B)

    x0 = _bn_in(x, g_in, b_in)
    p = _sc_segsum(x0, src, dst)
    u, s1 = _c1(x0, p, W1, b1)
    v, s2 = _c2(u, s1, bn1_g, bn1_b, W2, b2)
    return _c3(v, s2, bno_g, bno_b, x0, Wq, bq, Wk, bk, Wv, bv)


# speedup vs baseline: 7.9809x; 1.0994x over previous
import functools

import jax
import jax.numpy as jnp
from jax import lax
from jax.experimental import pallas as pl
from jax.experimental.pallas import tpu as pltpu
from jax.experimental.pallas import tpu_sc as plsc

N = 10000
E = 320000
D = 128
EPS = 1e-5

NC = 2
NS = 16
NW = NC * NS
EPW = E // NW
B = 100
NB = EPW // B
NBC = 10
NCH = NB // NBC
NP = 10240
RPT = NP // NS
RCH = 32
NRC = RPT // RCH


def _sc_segsum_body(x0_hbm, src_hbm, dst_hbm, out_hbm,
                    src_v, dst_v, rows, stage, acc_sh, gsem, ssem):
    cid = lax.axis_index("c")
    sid = lax.axis_index("s")
    w = cid * NS + sid

    zero = jnp.zeros((16,), jnp.float32)

    for t in range(3):
        for rr in range(RCH):
            for ll in range(D // 16):
                stage[rr, pl.ds(ll * 16, 16)] = zero
        for r in range(NRC):
            rs = sid * RPT + r * RCH
            pltpu.sync_copy(stage, acc_sh.at[pl.ds(rs, RCH)])
        plsc.subcore_barrier()

        def chunk_body(c, carry):
            pltpu.sync_copy(src_hbm.at[t, w, c], src_v)
            pltpu.sync_copy(dst_hbm.at[t, w, c], dst_v)

            cp = {}
            sc = {}
            cp[0] = pltpu.async_copy(x0_hbm.at[src_v.at[0]], rows.at[0],
                                     gsem.at[0])
            cp[1] = pltpu.async_copy(x0_hbm.at[src_v.at[1]], rows.at[1],
                                     gsem.at[1])
            for j in range(NBC):
                cp[j].wait()
                sc[j] = pltpu.async_copy(rows.at[j % 3],
                                         acc_sh.at[dst_v.at[j]],
                                         ssem.at[j % 3], add=True)
                jn = j + 2
                if jn < NBC:
                    if jn >= 3:
                        sc[jn - 3].wait()
                    cp[jn] = pltpu.async_copy(x0_hbm.at[src_v.at[jn]],
                                              rows.at[jn % 3],
                                              gsem.at[jn % 3])
            for j in range(NBC - 3, NBC):
                sc[j].wait()
            return carry

        lax.fori_loop(0, NCH, chunk_body, 0)
        plsc.subcore_barrier()

        for r in range(NRC):
            rs = sid * RPT + r * RCH
            pltpu.sync_copy(acc_sh.at[pl.ds(rs, RCH)], stage)
            pltpu.sync_copy(stage, out_hbm.at[t, cid, pl.ds(rs, RCH)])
        plsc.subcore_barrier()


_sc_segsum = functools.partial(
    pl.kernel,
    out_type=jax.ShapeDtypeStruct((3, NC, NP, D), jnp.float32),
    mesh=plsc.VectorSubcoreMesh(core_axis_name="c", subcore_axis_name="s",
                                num_cores=NC, num_subcores=NS),
    scratch_types=[
        pltpu.VMEM((NBC, B), jnp.int32),
        pltpu.VMEM((NBC, B), jnp.int32),
        pltpu.VMEM((3, B, D), jnp.float32),
        pltpu.VMEM((RCH, D), jnp.float32),
        pltpu.VMEM_SHARED((NP, D), jnp.float32),
        pltpu.SemaphoreType.DMA((3,)),
        pltpu.SemaphoreType.DMA((3,)),
    ],
)(_sc_segsum_body)


def _bn_in_body(x_ref, g_ref, b_ref, o_ref):
    xv = x_ref[...]
    m = jnp.mean(xv, axis=0, keepdims=True)
    v = jnp.mean(jnp.square(xv - m), axis=0, keepdims=True)
    x0 = (xv - m) * lax.rsqrt(v + EPS) * g_ref[...] + b_ref[...]
    o_ref[...] = jnp.concatenate(
        [x0, jnp.zeros((NP - N, D), jnp.float32)], axis=0)


def _bn_in(x, g, b):
    return pl.pallas_call(
        _bn_in_body,
        out_shape=jax.ShapeDtypeStruct((NP, D), jnp.float32),
    )(x, g.reshape(1, D), b.reshape(1, D))


RB = 2000
G = N // RB


def _c1_body(x0_ref, p_ref, w1_ref, b1_ref, u_ref, s_ref, acc):
    i = pl.program_id(0)

    @pl.when(i == 0)
    def _():
        acc[...] = jnp.zeros_like(acc)

    for t in range(3):
        h = p_ref[t, 0] + p_ref[t, 1] + x0_ref[...]
        u = jnp.dot(h, w1_ref[t], preferred_element_type=jnp.float32) + b1_ref[t]
        u_ref[t] = u
        acc[2 * t:2 * t + 1, :] += jnp.sum(u, axis=0, keepdims=True)
        acc[2 * t + 1:2 * t + 2, :] += jnp.sum(u * u, axis=0, keepdims=True)

    @pl.when(i == G - 1)
    def _():
        s_ref[...] = acc[...]


def _c1(x0, p, W1, b1):
    return pl.pallas_call(
        _c1_body,
        grid=(G,),
        in_specs=[
            pl.BlockSpec((RB, D), lambda i: (i, 0)),
            pl.BlockSpec((3, NC, RB, D), lambda i: (0, 0, i, 0)),
            pl.BlockSpec((3, D, D), lambda i: (0, 0, 0)),
            pl.BlockSpec((3, 1, D), lambda i: (0, 0, 0)),
        ],
        out_specs=[
            pl.BlockSpec((3, RB, D), lambda i: (0, i, 0)),
            pl.BlockSpec((6, D), lambda i: (0, 0)),
        ],
        out_shape=[
            jax.ShapeDtypeStruct((3, N, D), jnp.float32),
            jax.ShapeDtypeStruct((6, D), jnp.float32),
        ],
        scratch_shapes=[pltpu.VMEM((6, D), jnp.float32)],
    )(x0, p, W1, b1.reshape(3, 1, D))


def _c2_body(u_ref, s1_ref, g1_ref, b1_ref, w2_ref, b2_ref, v_ref, s_ref, acc):
    i = pl.program_id(0)

    @pl.when(i == 0)
    def _():
        acc[...] = jnp.zeros_like(acc)

    for t in range(3):
        m = s1_ref[2 * t:2 * t + 1, :] * (1.0 / N)
        var = s1_ref[2 * t + 1:2 * t + 2, :] * (1.0 / N) - m * m
        a = (u_ref[t] - m) * lax.rsqrt(var + EPS) * g1_ref[t] + b1_ref[t]
        a = jnp.maximum(a, 0.0)
        v = jnp.dot(a, w2_ref[t], preferred_element_type=jnp.float32) + b2_ref[t]
        v_ref[t] = v
        acc[2 * t:2 * t + 1, :] += jnp.sum(v, axis=0, keepdims=True)
        acc[2 * t + 1:2 * t + 2, :] += jnp.sum(v * v, axis=0, keepdims=True)

    @pl.when(i == G - 1)
    def _():
        s_ref[...] = acc[...]


def _c2(u, s1, bn1_g, bn1_b, W2, b2):
    return pl.pallas_call(
        _c2_body,
        grid=(G,),
        in_specs=[
            pl.BlockSpec((3, RB, D), lambda i: (0, i, 0)),
            pl.BlockSpec((6, D), lambda i: (0, 0)),
            pl.BlockSpec((3, 1, D), lambda i: (0, 0, 0)),
            pl.BlockSpec((3, 1, D), lambda i: (0, 0, 0)),
            pl.BlockSpec((3, D, D), lambda i: (0, 0, 0)),
            pl.BlockSpec((3, 1, D), lambda i: (0, 0, 0)),
        ],
        out_specs=[
            pl.BlockSpec((3, RB, D), lambda i: (0, i, 0)),
            pl.BlockSpec((6, D), lambda i: (0, 0)),
        ],
        out_shape=[
            jax.ShapeDtypeStruct((3, N, D), jnp.float32),
            jax.ShapeDtypeStruct((6, D), jnp.float32),
        ],
        scratch_shapes=[pltpu.VMEM((6, D), jnp.float32)],
    )(u, s1, bn1_g.reshape(3, 1, D), bn1_b.reshape(3, 1, D), W2,
      b2.reshape(3, 1, D))


def _c3_body(v_ref, s2_ref, go_ref, bo_ref, x0_ref,
             wq_ref, bq_ref, wk_ref, bk_ref, wv_ref, bv_ref, o_ref):
    q = jnp.tanh(jnp.dot(x0_ref[...], wq_ref[...],
                         preferred_element_type=jnp.float32) + bq_ref[...])
    scores = []
    vals = []
    for t in range(3):
        m = s2_ref[2 * t:2 * t + 1, :] * (1.0 / N)
        var = s2_ref[2 * t + 1:2 * t + 2, :] * (1.0 / N) - m * m
        e = jnp.tanh((v_ref[t] - m) * lax.rsqrt(var + EPS) * go_ref[t]
                     + bo_ref[t])
        k = jnp.tanh(jnp.dot(e, wk_ref[...],
                             preferred_element_type=jnp.float32) + bk_ref[...])
        vv = jnp.tanh(jnp.dot(e, wv_ref[...],
                              preferred_element_type=jnp.float32) + bv_ref[...])
        scores.append(jnp.sum(k * q, axis=1, keepdims=True))
        vals.append(vv)
    smax = jnp.maximum(jnp.maximum(scores[0], scores[1]), scores[2])
    ew = [jnp.exp(s - smax) for s in scores]
    z = ew[0] + ew[1] + ew[2]
    o_ref[...] = (ew[0] * vals[0] + ew[1] * vals[1] + ew[2] * vals[2]) / z


def _c3(v, s2, bno_g, bno_b, x0, Wq, bq, Wk, bk, Wv, bv):
    return pl.pallas_call(
        _c3_body,
        grid=(G,),
        in_specs=[
            pl.BlockSpec((3, RB, D), lambda i: (0, i, 0)),
            pl.BlockSpec((6, D), lambda i: (0, 0)),
            pl.BlockSpec((3, 1, D), lambda i: (0, 0, 0)),
            pl.BlockSpec((3, 1, D), lambda i: (0, 0, 0)),
            pl.BlockSpec((RB, D), lambda i: (i, 0)),
            pl.BlockSpec((D, D), lambda i: (0, 0)),
            pl.BlockSpec((1, D), lambda i: (0, 0)),
            pl.BlockSpec((D, D), lambda i: (0, 0)),
            pl.BlockSpec((1, D), lambda i: (0, 0)),
            pl.BlockSpec((D, D), lambda i: (0, 0)),
            pl.BlockSpec((1, D), lambda i: (0, 0)),
        ],
        out_specs=pl.BlockSpec((RB, D), lambda i: (i, 0)),
        out_shape=jax.ShapeDtypeStruct((N, D), jnp.float32),
    )(v, s2, bno_g.reshape(3, 1, D), bno_b.reshape(3, 1, D), x0,
      Wq, bq.reshape(1, D), Wk, bk.reshape(1, D), Wv, bv.reshape(1, D))


def kernel(x, edge_index_p, edge_index_s, edge_index_v, g_in, b_in,
           W1, b1, bn1_g, bn1_b, W2, b2, bno_g, bno_b,
           Wq, bq, Wk, bk, Wv, bv):
    src = jnp.stack([edge_index_p[0], edge_index_s[0], edge_index_v[0]])
    dst = jnp.stack([edge_index_p[1], edge_index_s[1], edge_index_v[1]])
    src = src.reshape(3, NW, NCH, NBC, B)
    dst = dst.reshape(3, NW, NCH, NBC, B)

    x0 = _bn_in(x, g_in, b_in)
    p = _sc_segsum(x0, src, dst)
    u, s1 = _c1(x0, p, W1, b1)
    v, s2 = _c2(u, s1, bn1_g, bn1_b, W2, b2)
    return _c3(v, s2, bno_g, bno_b, x0, Wq, bq, Wk, bk, Wv, bv)


# speedup vs baseline: 8.2764x; 1.0370x over previous
import functools

import jax
import jax.numpy as jnp
from jax import lax
from jax.experimental import pallas as pl
from jax.experimental.pallas import tpu as pltpu
from jax.experimental.pallas import tpu_sc as plsc

N = 10000
E = 320000
D = 128
EPS = 1e-5

NC = 2
NS = 16
NW = NC * NS
EPW = E // NW
B = 100
NB = EPW // B
NBC = 10
NCH = NB // NBC
NP = 10240
RPT = NP // NS
RCH = 32
NRC = RPT // RCH


def _sc_segsum_body(x0_hbm, src_hbm, dst_hbm, out_hbm,
                    src_v, dst_v, rows, stage, acc_sh, gsem, ssem):
    cid = lax.axis_index("c")
    sid = lax.axis_index("s")
    w = cid * NS + sid

    zero = jnp.zeros((16,), jnp.float32)
    for rr in range(RCH):
        for ll in range(D // 16):
            stage[rr, pl.ds(ll * 16, 16)] = zero
    for r in range(NRC):
        rs = sid * RPT + r * RCH
        pltpu.sync_copy(stage, acc_sh.at[pl.ds(rs, RCH)])
    plsc.subcore_barrier()

    def chunk_body(c, carry):
        pltpu.sync_copy(src_hbm.at[w, c], src_v)
        pltpu.sync_copy(dst_hbm.at[w, c], dst_v)

        cp = {}
        sc = {}
        cp[0] = pltpu.async_copy(x0_hbm.at[src_v.at[0]], rows.at[0],
                                 gsem.at[0])
        cp[1] = pltpu.async_copy(x0_hbm.at[src_v.at[1]], rows.at[1],
                                 gsem.at[1])
        for j in range(NBC):
            cp[j].wait()
            sc[j] = pltpu.async_copy(rows.at[j % 3],
                                     acc_sh.at[dst_v.at[j]],
                                     ssem.at[j % 3], add=True)
            jn = j + 2
            if jn < NBC:
                if jn >= 3:
                    sc[jn - 3].wait()
                cp[jn] = pltpu.async_copy(x0_hbm.at[src_v.at[jn]],
                                          rows.at[jn % 3],
                                          gsem.at[jn % 3])
        for j in range(NBC - 3, NBC):
            sc[j].wait()
        return carry

    lax.fori_loop(0, NCH, chunk_body, 0)
    plsc.subcore_barrier()

    for r in range(NRC):
        rs = sid * RPT + r * RCH
        pltpu.sync_copy(acc_sh.at[pl.ds(rs, RCH)], stage)
        pltpu.sync_copy(stage, out_hbm.at[cid, pl.ds(rs, RCH)])


_sc_segsum = functools.partial(
    pl.kernel,
    out_type=jax.ShapeDtypeStruct((NC, NP, D), jnp.float32),
    mesh=plsc.VectorSubcoreMesh(core_axis_name="c", subcore_axis_name="s",
                                num_cores=NC, num_subcores=NS),
    scratch_types=[
        pltpu.VMEM((NBC, B), jnp.int32),
        pltpu.VMEM((NBC, B), jnp.int32),
        pltpu.VMEM((3, B, D), jnp.float32),
        pltpu.VMEM((RCH, D), jnp.float32),
        pltpu.VMEM_SHARED((NP, D), jnp.float32),
        pltpu.SemaphoreType.DMA((3,)),
        pltpu.SemaphoreType.DMA((3,)),
    ],
)(_sc_segsum_body)


def _bn_in_body(x_ref, g_ref, b_ref, o_ref):
    xv = x_ref[...]
    m = jnp.mean(xv, axis=0, keepdims=True)
    v = jnp.mean(jnp.square(xv - m), axis=0, keepdims=True)
    x0 = (xv - m) * lax.rsqrt(v + EPS) * g_ref[...] + b_ref[...]
    o_ref[...] = jnp.concatenate(
        [x0, jnp.zeros((NP - N, D), jnp.float32)], axis=0)


def _bn_in(x, g, b):
    return pl.pallas_call(
        _bn_in_body,
        out_shape=jax.ShapeDtypeStruct((NP, D), jnp.float32),
    )(x, g.reshape(1, D), b.reshape(1, D))


RB = 2000
G = N // RB


def _q_body(x0_ref, wq_ref, bq_ref, o_ref):
    o_ref[...] = jnp.tanh(
        jnp.dot(x0_ref[...], wq_ref[...], preferred_element_type=jnp.float32)
        + bq_ref[...])


def _q_proj(x0, Wq, bq):
    return pl.pallas_call(
        _q_body,
        grid=(G,),
        in_specs=[
            pl.BlockSpec((RB, D), lambda i: (i, 0)),
            pl.BlockSpec((D, D), lambda i: (0, 0)),
            pl.BlockSpec((1, D), lambda i: (0, 0)),
        ],
        out_specs=pl.BlockSpec((RB, D), lambda i: (i, 0)),
        out_shape=jax.ShapeDtypeStruct((N, D), jnp.float32),
    )(x0, Wq, bq.reshape(1, D))


def _c1_body(x0_ref, p_ref, w1_ref, b1_ref, u_ref, s_ref, acc):
    i = pl.program_id(0)

    @pl.when(i == 0)
    def _():
        acc[...] = jnp.zeros_like(acc)

    h = p_ref[0] + p_ref[1] + x0_ref[...]
    u = jnp.dot(h, w1_ref[...], preferred_element_type=jnp.float32) + b1_ref[...]
    u_ref[...] = u
    acc[0:1, :] += jnp.sum(u, axis=0, keepdims=True)
    acc[1:2, :] += jnp.sum(u * u, axis=0, keepdims=True)

    @pl.when(i == G - 1)
    def _():
        s_ref[...] = acc[...]


def _c1(x0, p, W1t, b1t):
    return pl.pallas_call(
        _c1_body,
        grid=(G,),
        in_specs=[
            pl.BlockSpec((RB, D), lambda i: (i, 0)),
            pl.BlockSpec((NC, RB, D), lambda i: (0, i, 0)),
            pl.BlockSpec((D, D), lambda i: (0, 0)),
            pl.BlockSpec((1, D), lambda i: (0, 0)),
        ],
        out_specs=[
            pl.BlockSpec((RB, D), lambda i: (i, 0)),
            pl.BlockSpec((2, D), lambda i: (0, 0)),
        ],
        out_shape=[
            jax.ShapeDtypeStruct((N, D), jnp.float32),
            jax.ShapeDtypeStruct((2, D), jnp.float32),
        ],
        scratch_shapes=[pltpu.VMEM((2, D), jnp.float32)],
    )(x0, p, W1t, b1t.reshape(1, D))


def _c2_body(u_ref, s1_ref, g1_ref, b1_ref, w2_ref, b2_ref, v_ref, s_ref, acc):
    i = pl.program_id(0)

    @pl.when(i == 0)
    def _():
        acc[...] = jnp.zeros_like(acc)

    m = s1_ref[0:1, :] * (1.0 / N)
    var = s1_ref[1:2, :] * (1.0 / N) - m * m
    a = (u_ref[...] - m) * lax.rsqrt(var + EPS) * g1_ref[...] + b1_ref[...]
    a = jnp.maximum(a, 0.0)
    v = jnp.dot(a, w2_ref[...], preferred_element_type=jnp.float32) + b2_ref[...]
    v_ref[...] = v
    acc[0:1, :] += jnp.sum(v, axis=0, keepdims=True)
    acc[1:2, :] += jnp.sum(v * v, axis=0, keepdims=True)

    @pl.when(i == G - 1)
    def _():
        s_ref[...] = acc[...]


def _c2(u, s1, g1t, b1t, W2t, b2t):
    return pl.pallas_call(
        _c2_body,
        grid=(G,),
        in_specs=[
            pl.BlockSpec((RB, D), lambda i: (i, 0)),
            pl.BlockSpec((2, D), lambda i: (0, 0)),
            pl.BlockSpec((1, D), lambda i: (0, 0)),
            pl.BlockSpec((1, D), lambda i: (0, 0)),
            pl.BlockSpec((D, D), lambda i: (0, 0)),
            pl.BlockSpec((1, D), lambda i: (0, 0)),
        ],
        out_specs=[
            pl.BlockSpec((RB, D), lambda i: (i, 0)),
            pl.BlockSpec((2, D), lambda i: (0, 0)),
        ],
        out_shape=[
            jax.ShapeDtypeStruct((N, D), jnp.float32),
            jax.ShapeDtypeStruct((2, D), jnp.float32),
        ],
        scratch_shapes=[pltpu.VMEM((2, D), jnp.float32)],
    )(u, s1, g1t.reshape(1, D), b1t.reshape(1, D), W2t, b2t.reshape(1, D))


def _c3_body(v0_ref, v1_ref, v2_ref, s0_ref, s1_ref, s2_ref,
             go_ref, bo_ref, q_ref, wk_ref, bk_ref, wv_ref, bv_ref, o_ref):
    q = q_ref[...]
    v_refs = (v0_ref, v1_ref, v2_ref)
    s_refs = (s0_ref, s1_ref, s2_ref)
    scores = []
    vals = []
    for t in range(3):
        m = s_refs[t][0:1, :] * (1.0 / N)
        var = s_refs[t][1:2, :] * (1.0 / N) - m * m
        e = jnp.tanh((v_refs[t][...] - m) * lax.rsqrt(var + EPS)
                     * go_ref[t] + bo_ref[t])
        k = jnp.tanh(jnp.dot(e, wk_ref[...],
                             preferred_element_type=jnp.float32) + bk_ref[...])
        vv = jnp.tanh(jnp.dot(e, wv_ref[...],
                              preferred_element_type=jnp.float32) + bv_ref[...])
        scores.append(jnp.sum(k * q, axis=1, keepdims=True))
        vals.append(vv)
    smax = jnp.maximum(jnp.maximum(scores[0], scores[1]), scores[2])
    ew = [jnp.exp(s - smax) for s in scores]
    z = ew[0] + ew[1] + ew[2]
    o_ref[...] = (ew[0] * vals[0] + ew[1] * vals[1] + ew[2] * vals[2]) / z


def _c3(vs, ss, bno_g, bno_b, q, Wk, bk, Wv, bv):
    blk = pl.BlockSpec((RB, D), lambda i: (i, 0))
    stat = pl.BlockSpec((2, D), lambda i: (0, 0))
    vecw = pl.BlockSpec((1, D), lambda i: (0, 0))
    matw = pl.BlockSpec((D, D), lambda i: (0, 0))
    return pl.pallas_call(
        _c3_body,
        grid=(G,),
        in_specs=[blk, blk, blk, stat, stat, stat,
                  pl.BlockSpec((3, 1, D), lambda i: (0, 0, 0)),
                  pl.BlockSpec((3, 1, D), lambda i: (0, 0, 0)),
                  blk, matw, vecw, matw, vecw],
        out_specs=blk,
        out_shape=jax.ShapeDtypeStruct((N, D), jnp.float32),
    )(vs[0], vs[1], vs[2], ss[0], ss[1], ss[2],
      bno_g.reshape(3, 1, D), bno_b.reshape(3, 1, D), q,
      Wk, bk.reshape(1, D), Wv, bv.reshape(1, D))


def kernel(x, edge_index_p, edge_index_s, edge_index_v, g_in, b_in,
           W1, b1, bn1_g, bn1_b, W2, b2, bno_g, bno_b,
           Wq, bq, Wk, bk, Wv, bv):
    x0 = _bn_in(x, g_in, b_in)
    q = _q_proj(x0, Wq, bq)

    edges = (edge_index_p, edge_index_s, edge_index_v)
    vs = []
    ss = []
    for t in range(3):
        src = edges[t][0].reshape(NW, NCH, NBC, B)
        dst = edges[t][1].reshape(NW, NCH, NBC, B)
        p = _sc_segsum(x0, src, dst)
        u, s1 = _c1(x0, p, W1[t], b1[t])
        v, s2 = _c2(u, s1, bn1_g[t], bn1_b[t], W2[t], b2[t])
        vs.append(v)
        ss.append(s2)
    return _c3(vs, ss, bno_g, bno_b, q, Wk, bk, Wv, bv)


# speedup vs baseline: 8.3745x; 1.0119x over previous
import functools

import jax
import jax.numpy as jnp
from jax import lax
from jax.experimental import pallas as pl
from jax.experimental.pallas import tpu as pltpu
from jax.experimental.pallas import tpu_sc as plsc

N = 10000
E = 320000
D = 128
EPS = 1e-5

NC = 2
NS = 16
NW = NC * NS
EPW = E // NW
B = 100
NB = EPW // B
NBC = 10
NCH = NB // NBC
NP = 10240
RPT = NP // NS
RCH = 32
NRC = RPT // RCH


def _sc_segsum_body(x0_hbm, src_hbm, dst_hbm, out_hbm,
                    src_v, dst_v, rows, stage, acc_sh, gsem, ssem):
    cid = lax.axis_index("c")
    sid = lax.axis_index("s")
    w = cid * NS + sid

    zero = jnp.zeros((16,), jnp.float32)
    for rr in range(RCH):
        for ll in range(D // 16):
            stage[rr, pl.ds(ll * 16, 16)] = zero
    for r in range(NRC):
        rs = sid * RPT + r * RCH
        pltpu.sync_copy(stage, acc_sh.at[pl.ds(rs, RCH)])
    plsc.subcore_barrier()

    def chunk_body(c, carry):
        pltpu.sync_copy(src_hbm.at[w, c], src_v)
        pltpu.sync_copy(dst_hbm.at[w, c], dst_v)

        cp = {}
        sc = {}
        cp[0] = pltpu.async_copy(x0_hbm.at[src_v.at[0]], rows.at[0],
                                 gsem.at[0])
        cp[1] = pltpu.async_copy(x0_hbm.at[src_v.at[1]], rows.at[1],
                                 gsem.at[1])
        for j in range(NBC):
            cp[j].wait()
            sc[j] = pltpu.async_copy(rows.at[j % 3],
                                     acc_sh.at[dst_v.at[j]],
                                     ssem.at[j % 3], add=True)
            jn = j + 2
            if jn < NBC:
                if jn >= 3:
                    sc[jn - 3].wait()
                cp[jn] = pltpu.async_copy(x0_hbm.at[src_v.at[jn]],
                                          rows.at[jn % 3],
                                          gsem.at[jn % 3])
        for j in range(NBC - 3, NBC):
            sc[j].wait()
        return carry

    lax.fori_loop(0, NCH, chunk_body, 0)
    plsc.subcore_barrier()

    for r in range(NRC):
        rs = sid * RPT + r * RCH
        pltpu.sync_copy(acc_sh.at[pl.ds(rs, RCH)], stage)
        pltpu.sync_copy(stage, out_hbm.at[cid, pl.ds(rs, RCH)])


_sc_segsum = functools.partial(
    pl.kernel,
    out_type=jax.ShapeDtypeStruct((NC, NP, D), jnp.float32),
    mesh=plsc.VectorSubcoreMesh(core_axis_name="c", subcore_axis_name="s",
                                num_cores=NC, num_subcores=NS),
    scratch_types=[
        pltpu.VMEM((NBC, B), jnp.int32),
        pltpu.VMEM((NBC, B), jnp.int32),
        pltpu.VMEM((3, B, D), jnp.float32),
        pltpu.VMEM((RCH, D), jnp.float32),
        pltpu.VMEM_SHARED((NP, D), jnp.float32),
        pltpu.SemaphoreType.DMA((3,)),
        pltpu.SemaphoreType.DMA((3,)),
    ],
)(_sc_segsum_body)


def _bn_in_body(x_ref, g_ref, b_ref, wq_ref, bq_ref, o_ref, q_ref):
    xv = x_ref[...]
    m = jnp.mean(xv, axis=0, keepdims=True)
    v = jnp.mean(jnp.square(xv - m), axis=0, keepdims=True)
    x0 = (xv - m) * lax.rsqrt(v + EPS) * g_ref[...] + b_ref[...]
    o_ref[...] = jnp.concatenate(
        [x0, jnp.zeros((NP - N, D), jnp.float32)], axis=0)
    q_ref[...] = jnp.tanh(
        jnp.dot(x0, wq_ref[...], preferred_element_type=jnp.float32)
        + bq_ref[...])


def _bn_in(x, g, b, Wq, bq):
    return pl.pallas_call(
        _bn_in_body,
        out_shape=[
            jax.ShapeDtypeStruct((NP, D), jnp.float32),
            jax.ShapeDtypeStruct((N, D), jnp.float32),
        ],
    )(x, g.reshape(1, D), b.reshape(1, D), Wq, bq.reshape(1, D))


RB = 2000
G = N // RB


def _c12_body(x0_ref, p_ref, w1_ref, b1_ref, g1_ref, bb1_ref, w2_ref,
              b2_ref, v_ref, s_ref, u_sc, st_sc):
    i = pl.program_id(0)

    @pl.when(i == 0)
    def _():
        st_sc[...] = jnp.zeros_like(st_sc)

    @pl.when(i < G)
    def _():
        h = p_ref[0] + p_ref[1] + x0_ref[...]
        u = (jnp.dot(h, w1_ref[...], preferred_element_type=jnp.float32)
             + b1_ref[...])
        u_sc[i] = u
        st_sc[0:1, :] += jnp.sum(u, axis=0, keepdims=True)
        st_sc[1:2, :] += jnp.sum(u * u, axis=0, keepdims=True)

    @pl.when(i >= G)
    def _():
        m = st_sc[0:1, :] * (1.0 / N)
        var = st_sc[1:2, :] * (1.0 / N) - m * m
        a = ((u_sc[i - G] - m) * lax.rsqrt(var + EPS) * g1_ref[...]
             + bb1_ref[...])
        a = jnp.maximum(a, 0.0)
        v = (jnp.dot(a, w2_ref[...], preferred_element_type=jnp.float32)
             + b2_ref[...])
        v_ref[...] = v
        st_sc[2:3, :] += jnp.sum(v, axis=0, keepdims=True)
        st_sc[3:4, :] += jnp.sum(v * v, axis=0, keepdims=True)

    @pl.when(i == 2 * G - 1)
    def _():
        s_ref[...] = st_sc[2:4, :]


def _c12(x0, p, W1t, b1t, g1t, bb1t, W2t, b2t):
    row = lambda i: (jnp.minimum(i, G - 1), 0)
    full = lambda i: (0, 0)
    return pl.pallas_call(
        _c12_body,
        grid=(2 * G,),
        in_specs=[
            pl.BlockSpec((RB, D), row),
            pl.BlockSpec((NC, RB, D), lambda i: (0, jnp.minimum(i, G - 1), 0)),
            pl.BlockSpec((D, D), full),
            pl.BlockSpec((1, D), full),
            pl.BlockSpec((1, D), full),
            pl.BlockSpec((1, D), full),
            pl.BlockSpec((D, D), full),
            pl.BlockSpec((1, D), full),
        ],
        out_specs=[
            pl.BlockSpec((RB, D), lambda i: (jnp.maximum(i - G, 0), 0)),
            pl.BlockSpec((2, D), full),
        ],
        out_shape=[
            jax.ShapeDtypeStruct((N, D), jnp.float32),
            jax.ShapeDtypeStruct((2, D), jnp.float32),
        ],
        scratch_shapes=[pltpu.VMEM((G, RB, D), jnp.float32),
                        pltpu.VMEM((4, D), jnp.float32)],
    )(x0, p, W1t, b1t.reshape(1, D), g1t.reshape(1, D), bb1t.reshape(1, D),
      W2t, b2t.reshape(1, D))


def _c3_body(v0_ref, v1_ref, v2_ref, s0_ref, s1_ref, s2_ref,
             go_ref, bo_ref, q_ref, wk_ref, bk_ref, wv_ref, bv_ref, o_ref):
    q = q_ref[...]
    v_refs = (v0_ref, v1_ref, v2_ref)
    s_refs = (s0_ref, s1_ref, s2_ref)
    scores = []
    vals = []
    for t in range(3):
        m = s_refs[t][0:1, :] * (1.0 / N)
        var = s_refs[t][1:2, :] * (1.0 / N) - m * m
        e = jnp.tanh((v_refs[t][...] - m) * lax.rsqrt(var + EPS)
                     * go_ref[t] + bo_ref[t])
        k = jnp.tanh(jnp.dot(e, wk_ref[...],
                             preferred_element_type=jnp.float32) + bk_ref[...])
        vv = jnp.tanh(jnp.dot(e, wv_ref[...],
                              preferred_element_type=jnp.float32) + bv_ref[...])
        scores.append(jnp.sum(k * q, axis=1, keepdims=True))
        vals.append(vv)
    smax = jnp.maximum(jnp.maximum(scores[0], scores[1]), scores[2])
    ew = [jnp.exp(s - smax) for s in scores]
    z = ew[0] + ew[1] + ew[2]
    o_ref[...] = (ew[0] * vals[0] + ew[1] * vals[1] + ew[2] * vals[2]) / z


def _c3(vs, ss, bno_g, bno_b, q, Wk, bk, Wv, bv):
    blk = pl.BlockSpec((RB, D), lambda i: (i, 0))
    stat = pl.BlockSpec((2, D), lambda i: (0, 0))
    vecw = pl.BlockSpec((1, D), lambda i: (0, 0))
    matw = pl.BlockSpec((D, D), lambda i: (0, 0))
    return pl.pallas_call(
        _c3_body,
        grid=(G,),
        in_specs=[blk, blk, blk, stat, stat, stat,
                  pl.BlockSpec((3, 1, D), lambda i: (0, 0, 0)),
                  pl.BlockSpec((3, 1, D), lambda i: (0, 0, 0)),
                  blk, matw, vecw, matw, vecw],
        out_specs=blk,
        out_shape=jax.ShapeDtypeStruct((N, D), jnp.float32),
    )(vs[0], vs[1], vs[2], ss[0], ss[1], ss[2],
      bno_g.reshape(3, 1, D), bno_b.reshape(3, 1, D), q,
      Wk, bk.reshape(1, D), Wv, bv.reshape(1, D))


def kernel(x, edge_index_p, edge_index_s, edge_index_v, g_in, b_in,
           W1, b1, bn1_g, bn1_b, W2, b2, bno_g, bno_b,
           Wq, bq, Wk, bk, Wv, bv):
    x0, q = _bn_in(x, g_in, b_in, Wq, bq)

    edges = (edge_index_p, edge_index_s, edge_index_v)
    vs = []
    ss = []
    for t in range(3):
        src = edges[t][0].reshape(NW, NCH, NBC, B)
        dst = edges[t][1].reshape(NW, NCH, NBC, B)
        p = _sc_segsum(x0, src, dst)
        v, s2 = _c12(x0, p, W1[t], b1[t], bn1_g[t], bn1_b[t], W2[t], b2[t])
        vs.append(v)
        ss.append(s2)
    return _c3(vs, ss, bno_g, bno_b, q, Wk, bk, Wv, bv)


# speedup vs baseline: 9.1092x; 1.0877x over previous
import functools

import jax
import jax.numpy as jnp
from jax import lax
from jax.experimental import pallas as pl
from jax.experimental.pallas import tpu as pltpu
from jax.experimental.pallas import tpu_sc as plsc

N = 10000
E = 320000
D = 128
EPS = 1e-5

NC = 2
NS = 16
NW = NC * NS
EPW = E // NW
B = 100
NB = EPW // B
NBC = 10
NCH = NB // NBC
NP = 10240
RPT = NP // NS
RCH = 8
NRC = RPT // RCH


def _sc_segsum_body(x0_hbm, src_hbm, dst_hbm, out_hbm,
                    srcA, dstA, srcB, dstB, rows, stage, acc_sh, gsem, ssem):
    cid = lax.axis_index("c")
    sid = lax.axis_index("s")
    w = cid * NS + sid
    NH = NCH // 2

    zero = jnp.zeros((16,), jnp.float32)
    for rr in range(RCH):
        for ll in range(D // 16):
            stage[rr, pl.ds(ll * 16, 16)] = zero
    for r in range(NRC):
        rs = sid * RPT + r * RCH
        pltpu.sync_copy(stage, acc_sh.at[pl.ds(rs, RCH)])
    plsc.subcore_barrier()

    pltpu.sync_copy(src_hbm.at[w, 0], srcA)
    pltpu.sync_copy(dst_hbm.at[w, 0], dstA)
    pltpu.async_copy(x0_hbm.at[srcA.at[0]], rows.at[0], gsem.at[0])
    pltpu.async_copy(x0_hbm.at[srcA.at[1]], rows.at[1], gsem.at[1])

    def pair_body(h, carry):
        c0 = 2 * h
        pltpu.sync_copy(src_hbm.at[w, c0 + 1], srcB)
        pltpu.sync_copy(dst_hbm.at[w, c0 + 1], dstB)

        cp = {}
        sc = {}
        for g in range(2 * NBC):
            half, j = divmod(g, NBC)
            sv = (srcA, srcB)[half]
            dv = (dstA, dstB)[half]
            buf = g % 3
            if g in cp:
                cp[g].wait()
            else:
                pltpu.make_async_copy(x0_hbm.at[sv.at[j]], rows.at[buf],
                                      gsem.at[buf]).wait()
            sc[g] = pltpu.async_copy(rows.at[buf], acc_sh.at[dv.at[j]],
                                     ssem.at[buf], add=True)
            gn = g + 2
            if gn < 2 * NBC:
                if gn >= 3:
                    sc[gn - 3].wait()
                half2, j2 = divmod(gn, NBC)
                sv2 = (srcA, srcB)[half2]
                cp[gn] = pltpu.async_copy(x0_hbm.at[sv2.at[j2]],
                                          rows.at[gn % 3],
                                          gsem.at[gn % 3])

        @pl.when(h + 1 < NH)
        def _():
            for b_ in range(3):
                pltpu.make_async_copy(rows.at[b_], acc_sh.at[dstB.at[0]],
                                      ssem.at[b_]).wait()
            pltpu.sync_copy(src_hbm.at[w, c0 + 2], srcA)
            pltpu.sync_copy(dst_hbm.at[w, c0 + 2], dstA)
            pltpu.async_copy(x0_hbm.at[srcA.at[0]], rows.at[0], gsem.at[0])
            pltpu.async_copy(x0_hbm.at[srcA.at[1]], rows.at[1], gsem.at[1])

        return carry

    lax.fori_loop(0, NH, pair_body, 0)
    for b_ in range(3):
        pltpu.make_async_copy(rows.at[b_], acc_sh.at[dstB.at[0]],
                              ssem.at[b_]).wait()
    plsc.subcore_barrier()

    fs = sid * RPT
    pltpu.sync_copy(acc_sh.at[pl.ds(fs, RPT)], out_hbm.at[cid, pl.ds(fs, RPT)])


_sc_segsum = functools.partial(
    pl.kernel,
    out_type=jax.ShapeDtypeStruct((NC, NP, D), jnp.float32),
    mesh=plsc.VectorSubcoreMesh(core_axis_name="c", subcore_axis_name="s",
                                num_cores=NC, num_subcores=NS),
    scratch_types=[
        pltpu.VMEM((NBC, B), jnp.int32),
        pltpu.VMEM((NBC, B), jnp.int32),
        pltpu.VMEM((NBC, B), jnp.int32),
        pltpu.VMEM((NBC, B), jnp.int32),
        pltpu.VMEM((3, B, D), jnp.float32),
        pltpu.VMEM((RCH, D), jnp.float32),
        pltpu.VMEM_SHARED((NP, D), jnp.float32),
        pltpu.SemaphoreType.DMA((3,)),
        pltpu.SemaphoreType.DMA((3,)),
    ],
)(_sc_segsum_body)


def _bn_in_body(x_ref, g_ref, b_ref, wq_ref, bq_ref, o_ref, q_ref):
    xv = x_ref[...]
    m = jnp.mean(xv, axis=0, keepdims=True)
    v = jnp.mean(jnp.square(xv - m), axis=0, keepdims=True)
    x0 = (xv - m) * lax.rsqrt(v + EPS) * g_ref[...] + b_ref[...]
    o_ref[...] = jnp.concatenate(
        [x0, jnp.zeros((NP - N, D), jnp.float32)], axis=0)
    q_ref[...] = jnp.tanh(
        jnp.dot(x0, wq_ref[...], preferred_element_type=jnp.float32)
        + bq_ref[...])


def _bn_in(x, g, b, Wq, bq):
    return pl.pallas_call(
        _bn_in_body,
        out_shape=[
            jax.ShapeDtypeStruct((NP, D), jnp.float32),
            jax.ShapeDtypeStruct((N, D), jnp.float32),
        ],
    )(x, g.reshape(1, D), b.reshape(1, D), Wq, bq.reshape(1, D))


RB = 2000
G = N // RB


def _c12_body(x0_ref, p_ref, w1_ref, b1_ref, g1_ref, bb1_ref, w2_ref,
              b2_ref, v_ref, s_ref, u_sc, st_sc):
    i = pl.program_id(0)

    @pl.when(i == 0)
    def _():
        st_sc[...] = jnp.zeros_like(st_sc)

    @pl.when(i < G)
    def _():
        h = p_ref[0] + p_ref[1] + x0_ref[...]
        u = (jnp.dot(h, w1_ref[...], preferred_element_type=jnp.float32)
             + b1_ref[...])
        u_sc[i] = u
        st_sc[0:1, :] += jnp.sum(u, axis=0, keepdims=True)
        st_sc[1:2, :] += jnp.sum(u * u, axis=0, keepdims=True)

    @pl.when(i >= G)
    def _():
        m = st_sc[0:1, :] * (1.0 / N)
        var = st_sc[1:2, :] * (1.0 / N) - m * m
        a = ((u_sc[i - G] - m) * lax.rsqrt(var + EPS) * g1_ref[...]
             + bb1_ref[...])
        a = jnp.maximum(a, 0.0)
        v = (jnp.dot(a, w2_ref[...], preferred_element_type=jnp.float32)
             + b2_ref[...])
        v_ref[...] = v
        st_sc[2:3, :] += jnp.sum(v, axis=0, keepdims=True)
        st_sc[3:4, :] += jnp.sum(v * v, axis=0, keepdims=True)

    @pl.when(i == 2 * G - 1)
    def _():
        s_ref[...] = st_sc[2:4, :]


def _c12(x0, p, W1t, b1t, g1t, bb1t, W2t, b2t):
    row = lambda i: (jnp.minimum(i, G - 1), 0)
    full = lambda i: (0, 0)
    return pl.pallas_call(
        _c12_body,
        grid=(2 * G,),
        in_specs=[
            pl.BlockSpec((RB, D), row),
            pl.BlockSpec((NC, RB, D), lambda i: (0, jnp.minimum(i, G - 1), 0)),
            pl.BlockSpec((D, D), full),
            pl.BlockSpec((1, D), full),
            pl.BlockSpec((1, D), full),
            pl.BlockSpec((1, D), full),
            pl.BlockSpec((D, D), full),
            pl.BlockSpec((1, D), full),
        ],
        out_specs=[
            pl.BlockSpec((RB, D), lambda i: (jnp.maximum(i - G, 0), 0)),
            pl.BlockSpec((2, D), full),
        ],
        out_shape=[
            jax.ShapeDtypeStruct((N, D), jnp.float32),
            jax.ShapeDtypeStruct((2, D), jnp.float32),
        ],
        scratch_shapes=[pltpu.VMEM((G, RB, D), jnp.float32),
                        pltpu.VMEM((4, D), jnp.float32)],
    )(x0, p, W1t, b1t.reshape(1, D), g1t.reshape(1, D), bb1t.reshape(1, D),
      W2t, b2t.reshape(1, D))


def _c3_body(v0_ref, v1_ref, v2_ref, s0_ref, s1_ref, s2_ref,
             go_ref, bo_ref, q_ref, wk_ref, bk_ref, wv_ref, bv_ref, o_ref):
    q = q_ref[...]
    v_refs = (v0_ref, v1_ref, v2_ref)
    s_refs = (s0_ref, s1_ref, s2_ref)
    scores = []
    vals = []
    for t in range(3):
        m = s_refs[t][0:1, :] * (1.0 / N)
        var = s_refs[t][1:2, :] * (1.0 / N) - m * m
        e = jnp.tanh((v_refs[t][...] - m) * lax.rsqrt(var + EPS)
                     * go_ref[t] + bo_ref[t])
        k = jnp.tanh(jnp.dot(e, wk_ref[...],
                             preferred_element_type=jnp.float32) + bk_ref[...])
        vv = jnp.tanh(jnp.dot(e, wv_ref[...],
                              preferred_element_type=jnp.float32) + bv_ref[...])
        scores.append(jnp.sum(k * q, axis=1, keepdims=True))
        vals.append(vv)
    smax = jnp.maximum(jnp.maximum(scores[0], scores[1]), scores[2])
    ew = [jnp.exp(s - smax) for s in scores]
    z = ew[0] + ew[1] + ew[2]
    o_ref[...] = (ew[0] * vals[0] + ew[1] * vals[1] + ew[2] * vals[2]) / z


def _c3(vs, ss, bno_g, bno_b, q, Wk, bk, Wv, bv):
    blk = pl.BlockSpec((RB, D), lambda i: (i, 0))
    stat = pl.BlockSpec((2, D), lambda i: (0, 0))
    vecw = pl.BlockSpec((1, D), lambda i: (0, 0))
    matw = pl.BlockSpec((D, D), lambda i: (0, 0))
    return pl.pallas_call(
        _c3_body,
        grid=(G,),
        in_specs=[blk, blk, blk, stat, stat, stat,
                  pl.BlockSpec((3, 1, D), lambda i: (0, 0, 0)),
                  pl.BlockSpec((3, 1, D), lambda i: (0, 0, 0)),
                  blk, matw, vecw, matw, vecw],
        out_specs=blk,
        out_shape=jax.ShapeDtypeStruct((N, D), jnp.float32),
    )(vs[0], vs[1], vs[2], ss[0], ss[1], ss[2],
      bno_g.reshape(3, 1, D), bno_b.reshape(3, 1, D), q,
      Wk, bk.reshape(1, D), Wv, bv.reshape(1, D))


def kernel(x, edge_index_p, edge_index_s, edge_index_v, g_in, b_in,
           W1, b1, bn1_g, bn1_b, W2, b2, bno_g, bno_b,
           Wq, bq, Wk, bk, Wv, bv):
    x0, q = _bn_in(x, g_in, b_in, Wq, bq)

    edges = (edge_index_p, edge_index_s, edge_index_v)
    vs = []
    ss = []
    for t in range(3):
        src = edges[t][0].reshape(NW, NCH, NBC, B)
        dst = edges[t][1].reshape(NW, NCH, NBC, B)
        p = _sc_segsum(x0, src, dst)
        v, s2 = _c12(x0, p, W1[t], b1[t], bn1_g[t], bn1_b[t], W2[t], b2[t])
        vs.append(v)
        ss.append(s2)
    return _c3(vs, ss, bno_g, bno_b, q, Wk, bk, Wv, bv)


# speedup vs baseline: 9.2955x; 1.0204x over previous
import functools

import jax
import jax.numpy as jnp
from jax import lax
from jax.experimental import pallas as pl
from jax.experimental.pallas import tpu as pltpu
from jax.experimental.pallas import tpu_sc as plsc

N = 10000
E = 320000
D = 128
EPS = 1e-5

NC = 2
NS = 16
NW = NC * NS
EPW = E // NW
B = 100
NB = EPW // B
NBC = 10
NCH = NB // NBC
NP = 10240
RPT = NP // NS
RCH = 8
NRC = RPT // RCH


def _sc_segsum_body(x0_hbm, src_hbm, dst_hbm, out_hbm,
                    srcA, dstA, srcB, dstB, rows, stage, acc_sh, gsem, ssem):
    cid = lax.axis_index("c")
    sid = lax.axis_index("s")
    w = cid * NS + sid
    NH = NCH // 2

    zero = jnp.zeros((16,), jnp.float32)
    for rr in range(RCH):
        for ll in range(D // 16):
            stage[rr, pl.ds(ll * 16, 16)] = zero
    for r in range(NRC):
        rs = sid * RPT + r * RCH
        pltpu.async_copy(stage, acc_sh.at[pl.ds(rs, RCH)], gsem.at[0])
    for r in range(NRC):
        rs = sid * RPT + r * RCH
        pltpu.make_async_copy(stage, acc_sh.at[pl.ds(rs, RCH)],
                              gsem.at[0]).wait()
    plsc.subcore_barrier()

    pltpu.sync_copy(src_hbm.at[w, 0], srcA)
    pltpu.sync_copy(dst_hbm.at[w, 0], dstA)
    pltpu.async_copy(x0_hbm.at[srcA.at[0]], rows.at[0], gsem.at[0])
    pltpu.async_copy(x0_hbm.at[srcA.at[1]], rows.at[1], gsem.at[1])

    def pair_body(h, carry):
        c0 = 2 * h
        pltpu.sync_copy(src_hbm.at[w, c0 + 1], srcB)
        pltpu.sync_copy(dst_hbm.at[w, c0 + 1], dstB)

        cp = {}
        sc = {}
        for g in range(2 * NBC):
            half, j = divmod(g, NBC)
            sv = (srcA, srcB)[half]
            dv = (dstA, dstB)[half]
            buf = g % 3
            if g in cp:
                cp[g].wait()
            else:
                pltpu.make_async_copy(x0_hbm.at[sv.at[j]], rows.at[buf],
                                      gsem.at[buf]).wait()
            sc[g] = pltpu.async_copy(rows.at[buf], acc_sh.at[dv.at[j]],
                                     ssem.at[buf], add=True)
            gn = g + 2
            if gn < 2 * NBC:
                if gn >= 3:
                    sc[gn - 3].wait()
                half2, j2 = divmod(gn, NBC)
                sv2 = (srcA, srcB)[half2]
                cp[gn] = pltpu.async_copy(x0_hbm.at[sv2.at[j2]],
                                          rows.at[gn % 3],
                                          gsem.at[gn % 3])

        @pl.when(h + 1 < NH)
        def _():
            for b_ in range(3):
                pltpu.make_async_copy(rows.at[b_], acc_sh.at[dstB.at[0]],
                                      ssem.at[b_]).wait()
            pltpu.sync_copy(src_hbm.at[w, c0 + 2], srcA)
            pltpu.sync_copy(dst_hbm.at[w, c0 + 2], dstA)
            pltpu.async_copy(x0_hbm.at[srcA.at[0]], rows.at[0], gsem.at[0])
            pltpu.async_copy(x0_hbm.at[srcA.at[1]], rows.at[1], gsem.at[1])

        return carry

    lax.fori_loop(0, NH, pair_body, 0)
    for b_ in range(3):
        pltpu.make_async_copy(rows.at[b_], acc_sh.at[dstB.at[0]],
                              ssem.at[b_]).wait()
    plsc.subcore_barrier()

    fs = sid * RPT
    pltpu.sync_copy(acc_sh.at[pl.ds(fs, RPT)], out_hbm.at[cid, pl.ds(fs, RPT)])


_sc_segsum = functools.partial(
    pl.kernel,
    out_type=jax.ShapeDtypeStruct((NC, NP, D), jnp.float32),
    mesh=plsc.VectorSubcoreMesh(core_axis_name="c", subcore_axis_name="s",
                                num_cores=NC, num_subcores=NS),
    scratch_types=[
        pltpu.VMEM((NBC, B), jnp.int32),
        pltpu.VMEM((NBC, B), jnp.int32),
        pltpu.VMEM((NBC, B), jnp.int32),
        pltpu.VMEM((NBC, B), jnp.int32),
        pltpu.VMEM((3, B, D), jnp.float32),
        pltpu.VMEM((RCH, D), jnp.float32),
        pltpu.VMEM_SHARED((NP, D), jnp.float32),
        pltpu.SemaphoreType.DMA((3,)),
        pltpu.SemaphoreType.DMA((3,)),
    ],
)(_sc_segsum_body)


def _bn_in_body(x_ref, g_ref, b_ref, wq_ref, bq_ref, o_ref, q_ref, st):
    i = pl.program_id(0)

    @pl.when(i == 0)
    def _():
        st[...] = jnp.zeros_like(st)

    @pl.when(i < GB)
    def _():
        xv = x_ref[...]
        st[0:1, :] += jnp.sum(xv, axis=0, keepdims=True)
        st[1:2, :] += jnp.sum(xv * xv, axis=0, keepdims=True)

    @pl.when(i >= GB)
    def _():
        m = st[0:1, :] * (1.0 / N)
        var = st[1:2, :] * (1.0 / N) - m * m
        x0 = (x_ref[...] - m) * lax.rsqrt(var + EPS) * g_ref[...] + b_ref[...]
        o_ref[...] = x0
        q_ref[...] = jnp.tanh(
            jnp.dot(x0, wq_ref[...], preferred_element_type=jnp.float32)
            + bq_ref[...])


def _bn_in(x, g, b, Wq, bq):
    row = lambda i: (lax.rem(i, GB), 0)
    full = lambda i: (0, 0)
    x0_pad, q = pl.pallas_call(
        _bn_in_body,
        grid=(2 * GB,),
        in_specs=[
            pl.BlockSpec((RBB, D), row),
            pl.BlockSpec((1, D), full),
            pl.BlockSpec((1, D), full),
            pl.BlockSpec((D, D), full),
            pl.BlockSpec((1, D), full),
        ],
        out_specs=[
            pl.BlockSpec((RBB, D), lambda i: (jnp.maximum(i - GB, 0), 0)),
            pl.BlockSpec((RBB, D), lambda i: (jnp.maximum(i - GB, 0), 0)),
        ],
        out_shape=[
            jax.ShapeDtypeStruct((NP, D), jnp.float32),
            jax.ShapeDtypeStruct((N, D), jnp.float32),
        ],
        scratch_shapes=[pltpu.VMEM((2, D), jnp.float32)],
    )(x, g.reshape(1, D), b.reshape(1, D), Wq, bq.reshape(1, D))
    return x0_pad, q


RBB = 2000
GB = N // RBB
RB = 2000
G = N // RB


def _c12_body(x0_ref, p_ref, w1_ref, b1_ref, g1_ref, bb1_ref, w2_ref,
              b2_ref, v_ref, s_ref, u_sc, st_sc):
    i = pl.program_id(0)

    @pl.when(i == 0)
    def _():
        st_sc[...] = jnp.zeros_like(st_sc)

    @pl.when(i < G)
    def _():
        h = p_ref[0] + p_ref[1] + x0_ref[...]
        u = (jnp.dot(h, w1_ref[...], preferred_element_type=jnp.float32)
             + b1_ref[...])
        u_sc[i] = u
        st_sc[0:1, :] += jnp.sum(u, axis=0, keepdims=True)
        st_sc[1:2, :] += jnp.sum(u * u, axis=0, keepdims=True)

    @pl.when(i >= G)
    def _():
        m = st_sc[0:1, :] * (1.0 / N)
        var = st_sc[1:2, :] * (1.0 / N) - m * m
        a = ((u_sc[i - G] - m) * lax.rsqrt(var + EPS) * g1_ref[...]
             + bb1_ref[...])
        a = jnp.maximum(a, 0.0)
        v = (jnp.dot(a, w2_ref[...], preferred_element_type=jnp.float32)
             + b2_ref[...])
        v_ref[...] = v
        st_sc[2:3, :] += jnp.sum(v, axis=0, keepdims=True)
        st_sc[3:4, :] += jnp.sum(v * v, axis=0, keepdims=True)

    @pl.when(i == 2 * G - 1)
    def _():
        s_ref[...] = st_sc[2:4, :]


def _c12(x0, p, W1t, b1t, g1t, bb1t, W2t, b2t):
    row = lambda i: (jnp.minimum(i, G - 1), 0)
    full = lambda i: (0, 0)
    return pl.pallas_call(
        _c12_body,
        grid=(2 * G,),
        in_specs=[
            pl.BlockSpec((RB, D), row),
            pl.BlockSpec((NC, RB, D), lambda i: (0, jnp.minimum(i, G - 1), 0)),
            pl.BlockSpec((D, D), full),
            pl.BlockSpec((1, D), full),
            pl.BlockSpec((1, D), full),
            pl.BlockSpec((1, D), full),
            pl.BlockSpec((D, D), full),
            pl.BlockSpec((1, D), full),
        ],
        out_specs=[
            pl.BlockSpec((RB, D), lambda i: (jnp.maximum(i - G, 0), 0)),
            pl.BlockSpec((2, D), full),
        ],
        out_shape=[
            jax.ShapeDtypeStruct((N, D), jnp.float32),
            jax.ShapeDtypeStruct((2, D), jnp.float32),
        ],
        scratch_shapes=[pltpu.VMEM((G, RB, D), jnp.float32),
                        pltpu.VMEM((4, D), jnp.float32)],
    )(x0, p, W1t, b1t.reshape(1, D), g1t.reshape(1, D), bb1t.reshape(1, D),
      W2t, b2t.reshape(1, D))


def _c3_body(v0_ref, v1_ref, v2_ref, s0_ref, s1_ref, s2_ref,
             go_ref, bo_ref, q_ref, wk_ref, bk_ref, wv_ref, bv_ref, o_ref):
    q = q_ref[...]
    v_refs = (v0_ref, v1_ref, v2_ref)
    s_refs = (s0_ref, s1_ref, s2_ref)
    scores = []
    vals = []
    for t in range(3):
        m = s_refs[t][0:1, :] * (1.0 / N)
        var = s_refs[t][1:2, :] * (1.0 / N) - m * m
        e = jnp.tanh((v_refs[t][...] - m) * lax.rsqrt(var + EPS)
                     * go_ref[t] + bo_ref[t])
        k = jnp.tanh(jnp.dot(e, wk_ref[...],
                             preferred_element_type=jnp.float32) + bk_ref[...])
        vv = jnp.tanh(jnp.dot(e, wv_ref[...],
                              preferred_element_type=jnp.float32) + bv_ref[...])
        scores.append(jnp.sum(k * q, axis=1, keepdims=True))
        vals.append(vv)
    smax = jnp.maximum(jnp.maximum(scores[0], scores[1]), scores[2])
    ew = [jnp.exp(s - smax) for s in scores]
    z = ew[0] + ew[1] + ew[2]
    o_ref[...] = (ew[0] * vals[0] + ew[1] * vals[1] + ew[2] * vals[2]) / z


def _c3(vs, ss, bno_g, bno_b, q, Wk, bk, Wv, bv):
    blk = pl.BlockSpec((RB, D), lambda i: (i, 0))
    stat = pl.BlockSpec((2, D), lambda i: (0, 0))
    vecw = pl.BlockSpec((1, D), lambda i: (0, 0))
    matw = pl.BlockSpec((D, D), lambda i: (0, 0))
    return pl.pallas_call(
        _c3_body,
        grid=(G,),
        in_specs=[blk, blk, blk, stat, stat, stat,
                  pl.BlockSpec((3, 1, D), lambda i: (0, 0, 0)),
                  pl.BlockSpec((3, 1, D), lambda i: (0, 0, 0)),
                  blk, matw, vecw, matw, vecw],
        out_specs=blk,
        out_shape=jax.ShapeDtypeStruct((N, D), jnp.float32),
    )(vs[0], vs[1], vs[2], ss[0], ss[1], ss[2],
      bno_g.reshape(3, 1, D), bno_b.reshape(3, 1, D), q,
      Wk, bk.reshape(1, D), Wv, bv.reshape(1, D))


def kernel(x, edge_index_p, edge_index_s, edge_index_v, g_in, b_in,
           W1, b1, bn1_g, bn1_b, W2, b2, bno_g, bno_b,
           Wq, bq, Wk, bk, Wv, bv):
    x0, q = _bn_in(x, g_in, b_in, Wq, bq)

    edges = (edge_index_p, edge_index_s, edge_index_v)
    vs = []
    ss = []
    for t in range(3):
        src = edges[t][0].reshape(NW, NCH, NBC, B)
        dst = edges[t][1].reshape(NW, NCH, NBC, B)
        p = _sc_segsum(x0, src, dst)
        v, s2 = _c12(x0, p, W1[t], b1[t], bn1_g[t], bn1_b[t], W2[t], b2[t])
        vs.append(v)
        ss.append(s2)
    return _c3(vs, ss, bno_g, bno_b, q, Wk, bk, Wv, bv)


# speedup vs baseline: 9.4194x; 1.0133x over previous
import functools

import jax
import jax.numpy as jnp
from jax import lax
from jax.experimental import pallas as pl
from jax.experimental.pallas import tpu as pltpu
from jax.experimental.pallas import tpu_sc as plsc

N = 10000
E = 320000
D = 128
EPS = 1e-5

NC = 2
NS = 16
NW = NC * NS
EPW = E // NW
B = 100
NB = EPW // B
NBC = 10
NCH = NB // NBC
NP = 10240
RPT = NP // NS
RCH = 8
NRC = RPT // RCH


def _sc_segsum_body(x0_hbm, src_hbm, dst_hbm, out_hbm,
                    srcA, dstA, srcB, dstB, rows, stage, acc_sh, gsem, ssem):
    cid = lax.axis_index("c")
    sid = lax.axis_index("s")
    w = cid * NS + sid
    NH = NCH // 2

    zero = jnp.zeros((16,), jnp.float32)
    for rr in range(RCH):
        for ll in range(D // 16):
            stage[rr, pl.ds(ll * 16, 16)] = zero
    for r in range(NRC):
        rs = sid * RPT + r * RCH
        pltpu.async_copy(stage, acc_sh.at[pl.ds(rs, RCH)], gsem.at[0])
    for r in range(NRC):
        rs = sid * RPT + r * RCH
        pltpu.make_async_copy(stage, acc_sh.at[pl.ds(rs, RCH)],
                              gsem.at[0]).wait()
    plsc.subcore_barrier()

    pltpu.sync_copy(src_hbm.at[w, 0], srcA)
    pltpu.sync_copy(dst_hbm.at[w, 0], dstA)
    pltpu.async_copy(x0_hbm.at[srcA.at[0]], rows.at[0], gsem.at[0])
    pltpu.async_copy(x0_hbm.at[srcA.at[1]], rows.at[1], gsem.at[1])

    def pair_body(h, carry):
        c0 = 2 * h
        pltpu.sync_copy(src_hbm.at[w, c0 + 1], srcB)
        pltpu.sync_copy(dst_hbm.at[w, c0 + 1], dstB)

        cp = {}
        sc = {}
        for g in range(2 * NBC):
            half, j = divmod(g, NBC)
            sv = (srcA, srcB)[half]
            dv = (dstA, dstB)[half]
            buf = g % 3
            if g in cp:
                cp[g].wait()
            else:
                pltpu.make_async_copy(x0_hbm.at[sv.at[j]], rows.at[buf],
                                      gsem.at[buf]).wait()
            sc[g] = pltpu.async_copy(rows.at[buf], acc_sh.at[dv.at[j]],
                                     ssem.at[buf], add=True)
            gn = g + 2
            if gn < 2 * NBC:
                if gn >= 3:
                    sc[gn - 3].wait()
                half2, j2 = divmod(gn, NBC)
                sv2 = (srcA, srcB)[half2]
                cp[gn] = pltpu.async_copy(x0_hbm.at[sv2.at[j2]],
                                          rows.at[gn % 3],
                                          gsem.at[gn % 3])

        @pl.when(h + 1 < NH)
        def _():
            for b_ in range(3):
                pltpu.make_async_copy(rows.at[b_], acc_sh.at[dstB.at[0]],
                                      ssem.at[b_]).wait()
            pltpu.sync_copy(src_hbm.at[w, c0 + 2], srcA)
            pltpu.sync_copy(dst_hbm.at[w, c0 + 2], dstA)
            pltpu.async_copy(x0_hbm.at[srcA.at[0]], rows.at[0], gsem.at[0])
            pltpu.async_copy(x0_hbm.at[srcA.at[1]], rows.at[1], gsem.at[1])

        return carry

    lax.fori_loop(0, NH, pair_body, 0)
    for b_ in range(3):
        pltpu.make_async_copy(rows.at[b_], acc_sh.at[dstB.at[0]],
                              ssem.at[b_]).wait()
    plsc.subcore_barrier()

    fs = sid * RPT
    pltpu.sync_copy(acc_sh.at[pl.ds(fs, RPT)], out_hbm.at[cid, pl.ds(fs, RPT)])


_sc_segsum = functools.partial(
    pl.kernel,
    out_type=jax.ShapeDtypeStruct((NC, NP, D), jnp.float32),
    mesh=plsc.VectorSubcoreMesh(core_axis_name="c", subcore_axis_name="s",
                                num_cores=NC, num_subcores=NS),
    scratch_types=[
        pltpu.VMEM((NBC, B), jnp.int32),
        pltpu.VMEM((NBC, B), jnp.int32),
        pltpu.VMEM((NBC, B), jnp.int32),
        pltpu.VMEM((NBC, B), jnp.int32),
        pltpu.VMEM((3, B, D), jnp.float32),
        pltpu.VMEM((RCH, D), jnp.float32),
        pltpu.VMEM_SHARED((NP, D), jnp.float32),
        pltpu.SemaphoreType.DMA((3,)),
        pltpu.SemaphoreType.DMA((3,)),
    ],
)(_sc_segsum_body)


def _bn_in_body(x_ref, g_ref, b_ref, wq_ref, bq_ref, o_ref, q_ref, st):
    i = pl.program_id(0)

    @pl.when(i == 0)
    def _():
        st[...] = jnp.zeros_like(st)

    @pl.when(i < GB)
    def _():
        xv = x_ref[...]
        st[0:1, :] += jnp.sum(xv, axis=0, keepdims=True)
        st[1:2, :] += jnp.sum(xv * xv, axis=0, keepdims=True)

    @pl.when(i >= GB)
    def _():
        m = st[0:1, :] * (1.0 / N)
        var = st[1:2, :] * (1.0 / N) - m * m
        x0 = (x_ref[...] - m) * lax.rsqrt(var + EPS) * g_ref[...] + b_ref[...]
        o_ref[...] = x0
        q_ref[...] = jnp.tanh(
            jnp.dot(x0, wq_ref[...], preferred_element_type=jnp.float32)
            + bq_ref[...])


def _bn_in(x, g, b, Wq, bq):
    row = lambda i: (lax.rem(i, GB), 0)
    full = lambda i: (0, 0)
    x0_pad, q = pl.pallas_call(
        _bn_in_body,
        grid=(2 * GB,),
        in_specs=[
            pl.BlockSpec((RBB, D), row),
            pl.BlockSpec((1, D), full),
            pl.BlockSpec((1, D), full),
            pl.BlockSpec((D, D), full),
            pl.BlockSpec((1, D), full),
        ],
        out_specs=[
            pl.BlockSpec((RBB, D), lambda i: (jnp.maximum(i - GB, 0), 0)),
            pl.BlockSpec((RBB, D), lambda i: (jnp.maximum(i - GB, 0), 0)),
        ],
        out_shape=[
            jax.ShapeDtypeStruct((NP, D), jnp.float32),
            jax.ShapeDtypeStruct((N, D), jnp.float32),
        ],
        scratch_shapes=[pltpu.VMEM((2, D), jnp.float32)],
    )(x, g.reshape(1, D), b.reshape(1, D), Wq, bq.reshape(1, D))
    return x0_pad, q


RBB = 2000
GB = N // RBB
RB = 2000
G = N // RB


def _c12_body(x0_ref, p_ref, w1_ref, b1_ref, g1_ref, bb1_ref, w2_ref,
              b2_ref, v_ref, s_ref, u_sc, st_sc):
    i = pl.program_id(0)

    @pl.when(i == 0)
    def _():
        st_sc[...] = jnp.zeros_like(st_sc)

    @pl.when(i < G)
    def _():
        h = p_ref[0] + p_ref[1] + x0_ref[...]
        u = (jnp.dot(h, w1_ref[...], preferred_element_type=jnp.float32)
             + b1_ref[...])
        u_sc[i] = u
        st_sc[0:1, :] += jnp.sum(u, axis=0, keepdims=True)
        st_sc[1:2, :] += jnp.sum(u * u, axis=0, keepdims=True)

    @pl.when(i >= G)
    def _():
        m = st_sc[0:1, :] * (1.0 / N)
        var = st_sc[1:2, :] * (1.0 / N) - m * m
        a = ((u_sc[i - G] - m) * lax.rsqrt(var + EPS) * g1_ref[...]
             + bb1_ref[...])
        a = jnp.maximum(a, 0.0)
        v = (jnp.dot(a, w2_ref[...], preferred_element_type=jnp.float32)
             + b2_ref[...])
        v_ref[...] = v
        st_sc[2:3, :] += jnp.sum(v, axis=0, keepdims=True)
        st_sc[3:4, :] += jnp.sum(v * v, axis=0, keepdims=True)

    @pl.when(i == 2 * G - 1)
    def _():
        s_ref[...] = st_sc[2:4, :]


def _c12(x0, p, W1t, b1t, g1t, bb1t, W2t, b2t):
    row = lambda i: (jnp.minimum(i, G - 1), 0)
    full = lambda i: (0, 0)
    return pl.pallas_call(
        _c12_body,
        grid=(2 * G,),
        in_specs=[
            pl.BlockSpec((RB, D), row),
            pl.BlockSpec((NC, RB, D), lambda i: (0, jnp.minimum(i, G - 1), 0)),
            pl.BlockSpec((D, D), full),
            pl.BlockSpec((1, D), full),
            pl.BlockSpec((1, D), full),
            pl.BlockSpec((1, D), full),
            pl.BlockSpec((D, D), full),
            pl.BlockSpec((1, D), full),
        ],
        out_specs=[
            pl.BlockSpec((RB, D), lambda i: (jnp.maximum(i - G, 0), 0)),
            pl.BlockSpec((2, D), full),
        ],
        out_shape=[
            jax.ShapeDtypeStruct((N, D), jnp.float32),
            jax.ShapeDtypeStruct((2, D), jnp.float32),
        ],
        scratch_shapes=[pltpu.VMEM((G, RB, D), jnp.float32),
                        pltpu.VMEM((4, D), jnp.float32)],
    )(x0, p, W1t, b1t.reshape(1, D), g1t.reshape(1, D), bb1t.reshape(1, D),
      W2t, b2t.reshape(1, D))


def _c12f_body(x0_ref, p_ref, w1_ref, b1_ref, g1_ref, bb1_ref, w2_ref,
               b2_ref, v0_ref, v1_ref, s0_ref, s1_ref, go_ref, bo_ref,
               q_ref, wk_ref, bk_ref, wv_ref, bv_ref,
               o_ref, u_sc, v_sc, st_sc):
    i = pl.program_id(0)

    @pl.when(i == 0)
    def _():
        st_sc[...] = jnp.zeros_like(st_sc)

    @pl.when(i < G)
    def _():
        h = p_ref[0] + p_ref[1] + x0_ref[...]
        u = (jnp.dot(h, w1_ref[...], preferred_element_type=jnp.float32)
             + b1_ref[...])
        u_sc[i] = u
        st_sc[0:1, :] += jnp.sum(u, axis=0, keepdims=True)
        st_sc[1:2, :] += jnp.sum(u * u, axis=0, keepdims=True)

    @pl.when(jnp.logical_and(i >= G, i < 2 * G))
    def _():
        m = st_sc[0:1, :] * (1.0 / N)
        var = st_sc[1:2, :] * (1.0 / N) - m * m
        a = ((u_sc[i - G] - m) * lax.rsqrt(var + EPS) * g1_ref[...]
             + bb1_ref[...])
        a = jnp.maximum(a, 0.0)
        v = (jnp.dot(a, w2_ref[...], preferred_element_type=jnp.float32)
             + b2_ref[...])
        v_sc[i - G] = v
        st_sc[2:3, :] += jnp.sum(v, axis=0, keepdims=True)
        st_sc[3:4, :] += jnp.sum(v * v, axis=0, keepdims=True)

    @pl.when(i >= 2 * G)
    def _():
        q = q_ref[...]
        scores = []
        vals = []
        for t in range(3):
            if t == 0:
                vt = v0_ref[...]
                m = s0_ref[0:1, :] * (1.0 / N)
                var = s0_ref[1:2, :] * (1.0 / N) - m * m
            elif t == 1:
                vt = v1_ref[...]
                m = s1_ref[0:1, :] * (1.0 / N)
                var = s1_ref[1:2, :] * (1.0 / N) - m * m
            else:
                vt = v_sc[i - 2 * G]
                m = st_sc[2:3, :] * (1.0 / N)
                var = st_sc[3:4, :] * (1.0 / N) - m * m
            e = jnp.tanh((vt - m) * lax.rsqrt(var + EPS) * go_ref[t]
                         + bo_ref[t])
            k = jnp.tanh(jnp.dot(e, wk_ref[...],
                                 preferred_element_type=jnp.float32)
                         + bk_ref[...])
            vv = jnp.tanh(jnp.dot(e, wv_ref[...],
                                  preferred_element_type=jnp.float32)
                          + bv_ref[...])
            scores.append(jnp.sum(k * q, axis=1, keepdims=True))
            vals.append(vv)
        smax = jnp.maximum(jnp.maximum(scores[0], scores[1]), scores[2])
        ew = [jnp.exp(sc_ - smax) for sc_ in scores]
        z = ew[0] + ew[1] + ew[2]
        o_ref[...] = (ew[0] * vals[0] + ew[1] * vals[1]
                      + ew[2] * vals[2]) / z


def _c12f(x0, p, W1t, b1t, g1t, bb1t, W2t, b2t, v0, v1, s0, s1,
          bno_g, bno_b, q, Wk, bk, Wv, bv):
    row1 = lambda i: (jnp.minimum(i, G - 1), 0)
    row3 = lambda i: (jnp.maximum(i - 2 * G, 0), 0)
    full = lambda i: (0, 0)
    full3 = lambda i: (0, 0, 0)
    return pl.pallas_call(
        _c12f_body,
        grid=(3 * G,),
        in_specs=[
            pl.BlockSpec((RB, D), row1),
            pl.BlockSpec((NC, RB, D), lambda i: (0, jnp.minimum(i, G - 1), 0)),
            pl.BlockSpec((D, D), full),
            pl.BlockSpec((1, D), full),
            pl.BlockSpec((1, D), full),
            pl.BlockSpec((1, D), full),
            pl.BlockSpec((D, D), full),
            pl.BlockSpec((1, D), full),
            pl.BlockSpec((RB, D), row3),
            pl.BlockSpec((RB, D), row3),
            pl.BlockSpec((2, D), full),
            pl.BlockSpec((2, D), full),
            pl.BlockSpec((3, 1, D), full3),
            pl.BlockSpec((3, 1, D), full3),
            pl.BlockSpec((RB, D), row3),
            pl.BlockSpec((D, D), full),
            pl.BlockSpec((1, D), full),
            pl.BlockSpec((D, D), full),
            pl.BlockSpec((1, D), full),
        ],
        out_specs=pl.BlockSpec((RB, D), row3),
        out_shape=jax.ShapeDtypeStruct((N, D), jnp.float32),
        scratch_shapes=[pltpu.VMEM((G, RB, D), jnp.float32),
                        pltpu.VMEM((G, RB, D), jnp.float32),
                        pltpu.VMEM((4, D), jnp.float32)],
    )(x0, p, W1t, b1t.reshape(1, D), g1t.reshape(1, D), bb1t.reshape(1, D),
      W2t, b2t.reshape(1, D), v0, v1, s0, s1,
      bno_g.reshape(3, 1, D), bno_b.reshape(3, 1, D), q,
      Wk, bk.reshape(1, D), Wv, bv.reshape(1, D))


def _c3_body(v0_ref, v1_ref, v2_ref, s0_ref, s1_ref, s2_ref,
             go_ref, bo_ref, q_ref, wk_ref, bk_ref, wv_ref, bv_ref, o_ref):
    q = q_ref[...]
    v_refs = (v0_ref, v1_ref, v2_ref)
    s_refs = (s0_ref, s1_ref, s2_ref)
    scores = []
    vals = []
    for t in range(3):
        m = s_refs[t][0:1, :] * (1.0 / N)
        var = s_refs[t][1:2, :] * (1.0 / N) - m * m
        e = jnp.tanh((v_refs[t][...] - m) * lax.rsqrt(var + EPS)
                     * go_ref[t] + bo_ref[t])
        k = jnp.tanh(jnp.dot(e, wk_ref[...],
                             preferred_element_type=jnp.float32) + bk_ref[...])
        vv = jnp.tanh(jnp.dot(e, wv_ref[...],
                              preferred_element_type=jnp.float32) + bv_ref[...])
        scores.append(jnp.sum(k * q, axis=1, keepdims=True))
        vals.append(vv)
    smax = jnp.maximum(jnp.maximum(scores[0], scores[1]), scores[2])
    ew = [jnp.exp(s - smax) for s in scores]
    z = ew[0] + ew[1] + ew[2]
    o_ref[...] = (ew[0] * vals[0] + ew[1] * vals[1] + ew[2] * vals[2]) / z


def _c3(vs, ss, bno_g, bno_b, q, Wk, bk, Wv, bv):
    blk = pl.BlockSpec((RB, D), lambda i: (i, 0))
    stat = pl.BlockSpec((2, D), lambda i: (0, 0))
    vecw = pl.BlockSpec((1, D), lambda i: (0, 0))
    matw = pl.BlockSpec((D, D), lambda i: (0, 0))
    return pl.pallas_call(
        _c3_body,
        grid=(G,),
        in_specs=[blk, blk, blk, stat, stat, stat,
                  pl.BlockSpec((3, 1, D), lambda i: (0, 0, 0)),
                  pl.BlockSpec((3, 1, D), lambda i: (0, 0, 0)),
                  blk, matw, vecw, matw, vecw],
        out_specs=blk,
        out_shape=jax.ShapeDtypeStruct((N, D), jnp.float32),
    )(vs[0], vs[1], vs[2], ss[0], ss[1], ss[2],
      bno_g.reshape(3, 1, D), bno_b.reshape(3, 1, D), q,
      Wk, bk.reshape(1, D), Wv, bv.reshape(1, D))


def kernel(x, edge_index_p, edge_index_s, edge_index_v, g_in, b_in,
           W1, b1, bn1_g, bn1_b, W2, b2, bno_g, bno_b,
           Wq, bq, Wk, bk, Wv, bv):
    x0, q = _bn_in(x, g_in, b_in, Wq, bq)

    edges = (edge_index_p, edge_index_s, edge_index_v)
    vs = []
    ss = []
    for t in range(3):
        src = edges[t][0].reshape(NW, NCH, NBC, B)
        dst = edges[t][1].reshape(NW, NCH, NBC, B)
        p = _sc_segsum(x0, src, dst)
        if t < 2:
            v, s2 = _c12(x0, p, W1[t], b1[t], bn1_g[t], bn1_b[t],
                         W2[t], b2[t])
            vs.append(v)
            ss.append(s2)
        else:
            return _c12f(x0, p, W1[2], b1[2], bn1_g[2], bn1_b[2],
                         W2[2], b2[2], vs[0], vs[1], ss[0], ss[1],
                         bno_g, bno_b, q, Wk, bk, Wv, bv)
